# Initial kernel scaffold; baseline (speedup 1.0000x reference)
import jax
import jax.numpy as jnp
from jax.experimental import pallas as pl


def kernel(lig_x, lig_pos, lig_edge_index, lig_edge_attr, pocket_x, pocket_pos, pocket_edge_index, pocket_edge_attr, t, lig_batch, pocket_batch, L_We1, L_be1, L_We2, L_be2, L_Wa, L_ba, L_Wn1, L_bn1, L_Wn2, L_bn2, L_Wc1, L_bc1, L_Wc2, P_We1, P_be1, P_We2, P_be2, P_Wa, P_ba, P_Wn1, P_bn1, P_Wn2, P_bn2, P_Wc1, P_bc1, P_Wc2, W_emb_l, b_emb_l, W_emb_p, b_emb_p, Wt1, bt1, Wt2, bt2, Wp, bp, Wo, bo):
    raise NotImplementedError("write your pallas kernel here")


# speedup vs baseline: 1.3354x; 1.3354x over previous
import functools

import jax
import jax.numpy as jnp
from jax import lax
from jax.experimental import pallas as pl
from jax.experimental.pallas import tpu as pltpu
from jax.experimental.pallas import tpu_sc as plsc

F32 = jnp.float32
H = 128
ED = 16
NT = 10
TDIM = 128
NBATCH = 64
WID = 144
PW = 16
NB = 1000
EB = 1024
NCORES = 2
NSUB = 16
NWORK = NCORES * NSUB
CH = 128

_HI = lax.Precision.HIGHEST


def _dot(a, b):
    return jnp.dot(a, b, precision=_HI)


def _silu(x):
    return x * jax.nn.sigmoid(x)


def _mesh():
    return plsc.VectorSubcoreMesh(core_axis_name="c", subcore_axis_name="s",
                                  num_cores=NCORES, num_subcores=NSUB)


def _emb_body(x_ref, w_ref, b_ref, o_ref):
    o_ref[...] = _dot(x_ref[...], w_ref[...]) + b_ref[...]


def _emb_call(x, w, b):
    n = x.shape[0]
    return pl.pallas_call(
        _emb_body,
        grid=(n // NB,),
        in_specs=[pl.BlockSpec((NB, x.shape[1]), lambda i: (i, 0)),
                  pl.BlockSpec(w.shape, lambda i: (0, 0)),
                  pl.BlockSpec(b.shape, lambda i: (0, 0))],
        out_specs=pl.BlockSpec((NB, w.shape[1]), lambda i: (i, 0)),
        out_shape=jax.ShapeDtypeStruct((n, w.shape[1]), F32),
    )(x, w, b)


def _emb_lig_body(x_ref, bt_ref, w_ref, b_ref, tc_ref, o_ref):
    oh = (bt_ref[...] == lax.broadcasted_iota(jnp.int32, (1, NBATCH), 1)
          ).astype(F32)
    o_ref[...] = (_dot(x_ref[...], w_ref[...]) + b_ref[...]
                  + _dot(oh, tc_ref[...]))


def _emb_lig_call(x, batch2d, w, b, tcond):
    n = x.shape[0]
    return pl.pallas_call(
        _emb_lig_body,
        grid=(n // NB,),
        in_specs=[pl.BlockSpec((NB, x.shape[1]), lambda i: (i, 0)),
                  pl.BlockSpec((NB, 1), lambda i: (i, 0)),
                  pl.BlockSpec(w.shape, lambda i: (0, 0)),
                  pl.BlockSpec(b.shape, lambda i: (0, 0)),
                  pl.BlockSpec(tcond.shape, lambda i: (0, 0))],
        out_specs=pl.BlockSpec((NB, H), lambda i: (i, 0)),
        out_shape=jax.ShapeDtypeStruct((n, H), F32),
    )(x, batch2d, w, b, tcond)


def _pool_body(h_ref, bt_ref, o_ref):
    i = pl.program_id(0)
    oh = (bt_ref[...] == lax.broadcasted_iota(jnp.int32, (1, NBATCH), 1)
          ).astype(F32)
    ssum = lax.dot_general(oh, h_ref[...], (((0,), (0,)), ((), ())),
                           precision=_HI)
    lane = lax.broadcasted_iota(jnp.int32, (NB, PW), 1)
    ones0 = (lane == 0).astype(F32)
    scnt = lax.dot_general(oh, ones0, (((0,), (0,)), ((), ())),
                           precision=_HI)

    @pl.when(i == 0)
    def _():
        o_ref[:, :H] = ssum
        o_ref[:, H:] = scnt

    @pl.when(i > 0)
    def _():
        o_ref[:, :H] += ssum
        o_ref[:, H:] += scnt


def _pool_call(h, batch2d):
    n = h.shape[0]
    return pl.pallas_call(
        _pool_body,
        grid=(n // NB,),
        in_specs=[pl.BlockSpec((NB, H), lambda i: (i, 0)),
                  pl.BlockSpec((NB, 1), lambda i: (i, 0))],
        out_specs=pl.BlockSpec((NBATCH, H + PW), lambda i: (0, 0)),
        out_shape=jax.ShapeDtypeStruct((NBATCH, H + PW), F32),
    )(h, batch2d)


def _temb_cond_body(t_ref, wt1, bt1, wt2, bt2, ps_ref, wp, bp, o_ref):
    t = t_ref[...].astype(F32)
    half = TDIM // 2
    k = lax.broadcasted_iota(jnp.int32, (1, half), 1).astype(F32)
    freqs = jnp.exp(-jnp.log(10000.0) * k / float(half))
    args = t * freqs
    temb = jnp.concatenate([jnp.sin(args), jnp.cos(args)], axis=1)
    temb = _silu(_dot(temb, wt1[...]) + bt1[...])
    temb = _dot(temb, wt2[...]) + bt2[...]
    ps = ps_ref[...]
    lane = lax.broadcasted_iota(jnp.int32, (NBATCH, PW), 1)
    cnt = jnp.sum(ps[:, H:] * (lane == 0).astype(F32), axis=1, keepdims=True)
    pooled = ps[:, :H] / (cnt + 1e-6)
    cond = _silu(_dot(pooled, wp[...]) + bp[...])
    o_ref[...] = temb + cond


def _temb_cond_call(t2d, wt1, bt1, wt2, bt2, psum, wp, bp):
    return pl.pallas_call(
        _temb_cond_body,
        out_shape=jax.ShapeDtypeStruct((NBATCH, H), F32),
    )(t2d, wt1, bt1, wt2, bt2, psum, wp, bp)


def _pre_body(h_ref, p_ref, ws, wd, be1, ts_ref, td_ref):
    hb = h_ref[...]
    ts_ref[:, :H] = _dot(hb, ws[...]) + be1[...]
    ts_ref[:, H:] = p_ref[...]
    td_ref[:, :H] = _dot(hb, wd[...])
    td_ref[:, H:] = p_ref[...]


def _pre_call(h, pos16, ws, wd, be1):
    n = h.shape[0]
    return pl.pallas_call(
        _pre_body,
        grid=(n // NB,),
        in_specs=[pl.BlockSpec((NB, H), lambda i: (i, 0)),
                  pl.BlockSpec((NB, PW), lambda i: (i, 0)),
                  pl.BlockSpec((H, H), lambda i: (0, 0)),
                  pl.BlockSpec((H, H), lambda i: (0, 0)),
                  pl.BlockSpec((1, H), lambda i: (0, 0))],
        out_specs=[pl.BlockSpec((NB, WID), lambda i: (i, 0)),
                   pl.BlockSpec((NB, WID), lambda i: (i, 0))],
        out_shape=[jax.ShapeDtypeStruct((n, WID), F32),
                   jax.ShapeDtypeStruct((n, WID), F32)],
    )(h, pos16, ws, wd, be1)


def _edge_body(ne_real, gs_ref, gd_ref, ea_ref, wea, wd2, we2, be2, wa, ba,
               wc1, bc1, wc2, o_ref):
    i = pl.program_id(0)
    gs = gs_ref[...]
    gd = gd_ref[...]
    rel = gs[:, H:] - gd[:, H:]
    d2 = jnp.sum(rel * rel, axis=1, keepdims=True)
    m1 = _silu(gs[:, :H] + gd[:, :H] + d2 * wd2[...]
               + _dot(ea_ref[...], wea[...]))
    m2 = _silu(_dot(m1, we2[...]) + be2[...])
    gate = jax.nn.sigmoid(
        jnp.sum(m2 * wa[...], axis=1, keepdims=True) + ba[...])
    m = m2 * gate
    c2 = _silu(_dot(m, wc1[...]) + bc1[...])
    c = jnp.sum(c2 * wc2[...], axis=1, keepdims=True)
    row = i * EB + lax.broadcasted_iota(jnp.int32, (EB, 1), 0)
    valid = (row < ne_real).astype(F32)
    lane = lax.broadcasted_iota(jnp.int32, (EB, PW), 1)
    deg1 = (lane == 3).astype(F32)
    o_ref[:, :H] = m * valid
    o_ref[:, H:] = (rel * c + deg1) * valid


def _edge_call(ne_real, gs, gd, ea, wea, wd2, we2, be2, wa, ba, wc1, bc1, wc2):
    e_pad = gs.shape[0]
    body = functools.partial(_edge_body, ne_real)
    return pl.pallas_call(
        body,
        grid=(e_pad // EB,),
        in_specs=[pl.BlockSpec((EB, WID), lambda i: (i, 0)),
                  pl.BlockSpec((EB, WID), lambda i: (i, 0)),
                  pl.BlockSpec((EB, ED), lambda i: (i, 0)),
                  pl.BlockSpec((ED, H), lambda i: (0, 0)),
                  pl.BlockSpec((1, H), lambda i: (0, 0)),
                  pl.BlockSpec((H, H), lambda i: (0, 0)),
                  pl.BlockSpec((1, H), lambda i: (0, 0)),
                  pl.BlockSpec((1, H), lambda i: (0, 0)),
                  pl.BlockSpec((1, 1), lambda i: (0, 0)),
                  pl.BlockSpec((H, H), lambda i: (0, 0)),
                  pl.BlockSpec((1, H), lambda i: (0, 0)),
                  pl.BlockSpec((1, H), lambda i: (0, 0))],
        out_specs=pl.BlockSpec((EB, WID), lambda i: (i, 0)),
        out_shape=jax.ShapeDtypeStruct((e_pad, WID), F32),
    )(gs, gd, ea, wea, wd2, we2, be2, wa, ba, wc1, bc1, wc2)


def _node_body(h_ref, p_ref, a0_ref, a1_ref, wn1h, wn1a, bn1, wn2, bn2,
               ho_ref, po_ref):
    h = h_ref[...]
    aggm = a0_ref[:, :H] + a1_ref[:, :H]
    agg16 = a0_ref[:, H:] + a1_ref[:, H:]
    lane = lax.broadcasted_iota(jnp.int32, (NB, PW), 1)
    deg = jnp.sum(agg16 * (lane == 3).astype(F32), axis=1, keepdims=True)
    relc = jnp.where(lane < 3, agg16, 0.0)
    u = _silu(_dot(h, wn1h[...]) + _dot(aggm, wn1a[...]) + bn1[...])
    ho_ref[...] = h + _dot(u, wn2[...]) + bn2[...]
    po_ref[...] = p_ref[...] + relc / (deg + 1.0)


def _node_call(h, pos16, a0, a1, wn1h, wn1a, bn1, wn2, bn2):
    n = h.shape[0]
    return pl.pallas_call(
        _node_body,
        grid=(n // NB,),
        in_specs=[pl.BlockSpec((NB, H), lambda i: (i, 0)),
                  pl.BlockSpec((NB, PW), lambda i: (i, 0)),
                  pl.BlockSpec((NB, WID), lambda i: (i, 0)),
                  pl.BlockSpec((NB, WID), lambda i: (i, 0)),
                  pl.BlockSpec((H, H), lambda i: (0, 0)),
                  pl.BlockSpec((H, H), lambda i: (0, 0)),
                  pl.BlockSpec((1, H), lambda i: (0, 0)),
                  pl.BlockSpec((H, H), lambda i: (0, 0)),
                  pl.BlockSpec((1, H), lambda i: (0, 0))],
        out_specs=[pl.BlockSpec((NB, H), lambda i: (i, 0)),
                   pl.BlockSpec((NB, PW), lambda i: (i, 0))],
        out_shape=[jax.ShapeDtypeStruct((n, H), F32),
                   jax.ShapeDtypeStruct((n, PW), F32)],
    )(h, pos16, a0, a1, wn1h, wn1a, bn1, wn2, bn2)


def _final_body(h_ref, wo, bo, p_ref, p0_ref, t_ref, c_ref):
    t_ref[...] = _dot(h_ref[...], wo[...]) + bo[...]
    c_ref[...] = p_ref[...] - p0_ref[...]


def _final_call(h, wo, bo, pos16, pos016):
    n = h.shape[0]
    return pl.pallas_call(
        _final_body,
        grid=(n // NB,),
        in_specs=[pl.BlockSpec((NB, H), lambda i: (i, 0)),
                  pl.BlockSpec((H, NT), lambda i: (0, 0)),
                  pl.BlockSpec((1, NT), lambda i: (0, 0)),
                  pl.BlockSpec((NB, PW), lambda i: (i, 0)),
                  pl.BlockSpec((NB, PW), lambda i: (i, 0))],
        out_specs=[pl.BlockSpec((NB, NT), lambda i: (i, 0)),
                   pl.BlockSpec((NB, PW), lambda i: (i, 0))],
        out_shape=[jax.ShapeDtypeStruct((n, NT), F32),
                   jax.ShapeDtypeStruct((n, PW), F32)],
    )(h, wo, bo, pos16, pos016)


@functools.lru_cache(maxsize=None)
def _gather_kernel(n_rows, e_pad):
    chunks = e_pad // (NWORK * CH)
    per_w = chunks * CH

    @functools.partial(
        pl.kernel,
        out_type=(jax.ShapeDtypeStruct((e_pad, WID), F32),
                  jax.ShapeDtypeStruct((e_pad, WID), F32)),
        mesh=_mesh(),
        scratch_types=[pltpu.VMEM((CH,), jnp.int32),
                       pltpu.VMEM((CH, WID), F32),
                       pltpu.VMEM((CH,), jnp.int32),
                       pltpu.VMEM((CH, WID), F32),
                       pltpu.SemaphoreType.DMA,
                       pltpu.SemaphoreType.DMA],
        compiler_params=pltpu.CompilerParams(use_tc_tiling_on_sc=False))
    def k(tsrc, tdst, srci, dsti, gs, gd, idx_s, row_s, idx_d, row_d,
          sem_s, sem_d):
        w = lax.axis_index("c") * NSUB + lax.axis_index("s")

        def body(i, carry):
            base = w * per_w + i * CH
            pltpu.sync_copy(srci.at[pl.ds(base, CH)], idx_s)
            pltpu.sync_copy(dsti.at[pl.ds(base, CH)], idx_d)
            cs = pltpu.async_copy(tsrc.at[idx_s], row_s, sem_s)
            cd = pltpu.async_copy(tdst.at[idx_d], row_d, sem_d)
            cs.wait()
            cd.wait()
            pltpu.sync_copy(row_s, gs.at[pl.ds(base, CH)])
            pltpu.sync_copy(row_d, gd.at[pl.ds(base, CH)])
            return carry

        lax.fori_loop(0, chunks, body, 0)

    return k


@functools.lru_cache(maxsize=None)
def _scatter_kernel(n_rows, e_pad):
    chunks = e_pad // (NWORK * CH)
    per_w = chunks * CH
    rpt = n_rows // NSUB
    oc = 5
    ocs = rpt // oc

    @functools.partial(
        pl.kernel,
        out_type=jax.ShapeDtypeStruct((NCORES, n_rows, WID), F32),
        mesh=_mesh(),
        scratch_types=[pltpu.VMEM_SHARED((n_rows, WID), F32),
                       pltpu.VMEM((CH,), jnp.int32),
                       pltpu.VMEM((CH, WID), F32),
                       pltpu.VMEM((ocs, WID), F32)],
        compiler_params=pltpu.CompilerParams(use_tc_tiling_on_sc=False))
    def k(mr, dsti, zrows, out, acc, idx, val, tmp):
        cid = lax.axis_index("c")
        sid = lax.axis_index("s")
        w = cid * NSUB + sid
        pltpu.sync_copy(zrows, acc.at[pl.ds(sid * rpt, rpt)])
        plsc.subcore_barrier()

        def body(i, carry):
            base = w * per_w + i * CH
            pltpu.sync_copy(dsti.at[pl.ds(base, CH)], idx)
            pltpu.sync_copy(mr.at[pl.ds(base, CH)], val)
            pltpu.sync_copy(val, acc.at[idx], add=True)
            return carry

        lax.fori_loop(0, chunks, body, 0)
        plsc.subcore_barrier()

        def flush(j, carry):
            s = sid * rpt + j * ocs
            pltpu.sync_copy(acc.at[pl.ds(s, ocs)], tmp)
            pltpu.sync_copy(tmp, out.at[cid, pl.ds(s, ocs)])
            return carry

        lax.fori_loop(0, oc, flush, 0)

    return k


def _egnn_layer(h, pos16, src_p, dst_p, ea_p, ne_real, zrows, p):
    n = h.shape[0]
    e_pad = src_p.shape[0]
    we1 = p['We1']
    tsrc, tdst = _pre_call(h, pos16, we1[:H], we1[H:2 * H],
                           p['be1'].reshape(1, H))
    gs, gd = _gather_kernel(n, e_pad)(tsrc, tdst, src_p, dst_p)
    mr = _edge_call(ne_real, gs, gd, ea_p,
                    we1[2 * H + 1:], we1[2 * H].reshape(1, H),
                    p['We2'], p['be2'].reshape(1, H),
                    p['Wa'].reshape(1, H), p['ba'].reshape(1, 1),
                    p['Wc1'], p['bc1'].reshape(1, H),
                    p['Wc2'].reshape(1, H))
    part = _scatter_kernel(n, e_pad)(mr, dst_p, zrows)
    wn1 = p['Wn1']
    return _node_call(h, pos16, part[0], part[1], wn1[:H], wn1[H:],
                      p['bn1'].reshape(1, H), p['Wn2'],
                      p['bn2'].reshape(1, H))


def _pad_edges(edge_index, edge_attr):
    e = edge_index.shape[1]
    e_pad = -(-e // (NWORK * CH)) * (NWORK * CH)
    pad = e_pad - e
    src = jnp.pad(edge_index[0].astype(jnp.int32), (0, pad))
    dst = jnp.pad(edge_index[1].astype(jnp.int32), (0, pad))
    ea = jnp.pad(edge_attr, ((0, pad), (0, 0)))
    return src, dst, ea, e


def kernel(lig_x, lig_pos, lig_edge_index, lig_edge_attr, pocket_x,
           pocket_pos, pocket_edge_index, pocket_edge_attr, t, lig_batch,
           pocket_batch, L_We1, L_be1, L_We2, L_be2, L_Wa, L_ba, L_Wn1,
           L_bn1, L_Wn2, L_bn2, L_Wc1, L_bc1, L_Wc2, P_We1, P_be1, P_We2,
           P_be2, P_Wa, P_ba, P_Wn1, P_bn1, P_Wn2, P_bn2, P_Wc1, P_bc1,
           P_Wc2, W_emb_l, b_emb_l, W_emb_p, b_emb_p, Wt1, bt1, Wt2, bt2,
           Wp, bp, Wo, bo):
    pnames = ['We1', 'be1', 'We2', 'be2', 'Wa', 'ba', 'Wn1', 'bn1',
              'Wn2', 'bn2', 'Wc1', 'bc1', 'Wc2']
    lw = dict(zip(pnames, [L_We1, L_be1, L_We2, L_be2, L_Wa, L_ba, L_Wn1,
                           L_bn1, L_Wn2, L_bn2, L_Wc1, L_bc1, L_Wc2]))
    pw = dict(zip(pnames, [P_We1, P_be1, P_We2, P_be2, P_Wa, P_ba, P_Wn1,
                           P_bn1, P_Wn2, P_bn2, P_Wc1, P_bc1, P_Wc2]))

    n_p = pocket_x.shape[0]
    n_l = lig_x.shape[0]
    zrows_p = jnp.zeros((n_p // NSUB, WID), F32)
    zrows_l = jnp.zeros((n_l // NSUB, WID), F32)

    hp = _emb_call(pocket_x, W_emb_p, b_emb_p.reshape(1, H))
    pp16 = jnp.pad(pocket_pos, ((0, 0), (0, PW - 3)))
    src_p, dst_p, ea_p, ne_p = _pad_edges(pocket_edge_index,
                                          pocket_edge_attr)
    for i in range(P_We1.shape[0]):
        hp, pp16 = _egnn_layer(hp, pp16, src_p, dst_p, ea_p, ne_p, zrows_p,
                               {k: v[i] for k, v in pw.items()})

    psum = _pool_call(hp, pocket_batch.astype(jnp.int32).reshape(n_p, 1))
    tcond = _temb_cond_call(t.astype(jnp.int32).reshape(NBATCH, 1),
                            Wt1, bt1.reshape(1, H), Wt2, bt2.reshape(1, H),
                            psum, Wp, bp.reshape(1, H))

    h = _emb_lig_call(lig_x, lig_batch.astype(jnp.int32).reshape(n_l, 1),
                      W_emb_l, b_emb_l.reshape(1, H), tcond)
    pl16 = jnp.pad(lig_pos, ((0, 0), (0, PW - 3)))
    src_l, dst_l, ea_l, ne_l = _pad_edges(lig_edge_index, lig_edge_attr)
    pos16 = pl16
    for i in range(L_We1.shape[0]):
        h, pos16 = _egnn_layer(h, pos16, src_l, dst_l, ea_l, ne_l, zrows_l,
                               {k: v[i] for k, v in lw.items()})

    type_pred, coord16 = _final_call(h, Wo, bo.reshape(1, NT), pos16, pl16)
    return type_pred, coord16[:, :3]


# speedup vs baseline: 1.3889x; 1.0401x over previous
import functools

import jax
import jax.numpy as jnp
from jax import lax
from jax.experimental import pallas as pl
from jax.experimental.pallas import tpu as pltpu
from jax.experimental.pallas import tpu_sc as plsc

F32 = jnp.float32
H = 128
ED = 16
NT = 10
TDIM = 128
NBATCH = 64
WID = 144
PW = 16
NB = 1000
EB = 1024
NCORES = 2
NSUB = 16
NWORK = NCORES * NSUB
CH = 128

_HI = lax.Precision.HIGHEST


def _dot(a, b):
    return jnp.dot(a, b, precision=_HI)


def _silu(x):
    return x * jax.nn.sigmoid(x)


def _mesh():
    return plsc.VectorSubcoreMesh(core_axis_name="c", subcore_axis_name="s",
                                  num_cores=NCORES, num_subcores=NSUB)


def _emb_body(x_ref, w_ref, b_ref, o_ref):
    o_ref[...] = _dot(x_ref[...], w_ref[...]) + b_ref[...]


def _emb_call(x, w, b):
    n = x.shape[0]
    return pl.pallas_call(
        _emb_body,
        grid=(n // NB,),
        in_specs=[pl.BlockSpec((NB, x.shape[1]), lambda i: (i, 0)),
                  pl.BlockSpec(w.shape, lambda i: (0, 0)),
                  pl.BlockSpec(b.shape, lambda i: (0, 0))],
        out_specs=pl.BlockSpec((NB, w.shape[1]), lambda i: (i, 0)),
        out_shape=jax.ShapeDtypeStruct((n, w.shape[1]), F32),
    )(x, w, b)


def _emb_lig_body(x_ref, bt_ref, w_ref, b_ref, tc_ref, o_ref):
    oh = (bt_ref[...] == lax.broadcasted_iota(jnp.int32, (1, NBATCH), 1)
          ).astype(F32)
    o_ref[...] = (_dot(x_ref[...], w_ref[...]) + b_ref[...]
                  + _dot(oh, tc_ref[...]))


def _emb_lig_call(x, batch2d, w, b, tcond):
    n = x.shape[0]
    return pl.pallas_call(
        _emb_lig_body,
        grid=(n // NB,),
        in_specs=[pl.BlockSpec((NB, x.shape[1]), lambda i: (i, 0)),
                  pl.BlockSpec((NB, 1), lambda i: (i, 0)),
                  pl.BlockSpec(w.shape, lambda i: (0, 0)),
                  pl.BlockSpec(b.shape, lambda i: (0, 0)),
                  pl.BlockSpec(tcond.shape, lambda i: (0, 0))],
        out_specs=pl.BlockSpec((NB, H), lambda i: (i, 0)),
        out_shape=jax.ShapeDtypeStruct((n, H), F32),
    )(x, batch2d, w, b, tcond)


def _pool_body(h_ref, bt_ref, o_ref):
    i = pl.program_id(0)
    oh = (bt_ref[...] == lax.broadcasted_iota(jnp.int32, (1, NBATCH), 1)
          ).astype(F32)
    ssum = lax.dot_general(oh, h_ref[...], (((0,), (0,)), ((), ())),
                           precision=_HI)
    lane = lax.broadcasted_iota(jnp.int32, (NB, PW), 1)
    ones0 = (lane == 0).astype(F32)
    scnt = lax.dot_general(oh, ones0, (((0,), (0,)), ((), ())),
                           precision=_HI)

    @pl.when(i == 0)
    def _():
        o_ref[:, :H] = ssum
        o_ref[:, H:] = scnt

    @pl.when(i > 0)
    def _():
        o_ref[:, :H] += ssum
        o_ref[:, H:] += scnt


def _pool_call(h, batch2d):
    n = h.shape[0]
    return pl.pallas_call(
        _pool_body,
        grid=(n // NB,),
        in_specs=[pl.BlockSpec((NB, H), lambda i: (i, 0)),
                  pl.BlockSpec((NB, 1), lambda i: (i, 0))],
        out_specs=pl.BlockSpec((NBATCH, H + PW), lambda i: (0, 0)),
        out_shape=jax.ShapeDtypeStruct((NBATCH, H + PW), F32),
    )(h, batch2d)


def _temb_cond_body(t_ref, wt1, bt1, wt2, bt2, ps_ref, wp, bp, o_ref):
    t = t_ref[...].astype(F32)
    half = TDIM // 2
    k = lax.broadcasted_iota(jnp.int32, (1, half), 1).astype(F32)
    freqs = jnp.exp(-jnp.log(10000.0) * k / float(half))
    args = t * freqs
    temb = jnp.concatenate([jnp.sin(args), jnp.cos(args)], axis=1)
    temb = _silu(_dot(temb, wt1[...]) + bt1[...])
    temb = _dot(temb, wt2[...]) + bt2[...]
    ps = ps_ref[...]
    lane = lax.broadcasted_iota(jnp.int32, (NBATCH, PW), 1)
    cnt = jnp.sum(ps[:, H:] * (lane == 0).astype(F32), axis=1, keepdims=True)
    pooled = ps[:, :H] / (cnt + 1e-6)
    cond = _silu(_dot(pooled, wp[...]) + bp[...])
    o_ref[...] = temb + cond


def _temb_cond_call(t2d, wt1, bt1, wt2, bt2, psum, wp, bp):
    return pl.pallas_call(
        _temb_cond_body,
        out_shape=jax.ShapeDtypeStruct((NBATCH, H), F32),
    )(t2d, wt1, bt1, wt2, bt2, psum, wp, bp)


def _pre_body(h_ref, p_ref, ws, wd, be1, ts_ref, td_ref):
    hb = h_ref[...]
    ts_ref[:, :H] = _dot(hb, ws[...]) + be1[...]
    ts_ref[:, H:] = p_ref[...]
    td_ref[:, :H] = _dot(hb, wd[...])
    td_ref[:, H:] = p_ref[...]


def _pre_call(h, pos16, ws, wd, be1):
    n = h.shape[0]
    return pl.pallas_call(
        _pre_body,
        grid=(n // NB,),
        in_specs=[pl.BlockSpec((NB, H), lambda i: (i, 0)),
                  pl.BlockSpec((NB, PW), lambda i: (i, 0)),
                  pl.BlockSpec((H, H), lambda i: (0, 0)),
                  pl.BlockSpec((H, H), lambda i: (0, 0)),
                  pl.BlockSpec((1, H), lambda i: (0, 0))],
        out_specs=[pl.BlockSpec((NB, WID), lambda i: (i, 0)),
                   pl.BlockSpec((NB, WID), lambda i: (i, 0))],
        out_shape=[jax.ShapeDtypeStruct((n, WID), F32),
                   jax.ShapeDtypeStruct((n, WID), F32)],
    )(h, pos16, ws, wd, be1)


def _edge_body(ne_real, gs_ref, gd_ref, ea_ref, wea, wd2, we2, be2, wa, ba,
               wc1, bc1, wc2, o_ref):
    i = pl.program_id(0)
    gs = gs_ref[...]
    gd = gd_ref[...]
    rel = gs[:, H:] - gd[:, H:]
    d2 = jnp.sum(rel * rel, axis=1, keepdims=True)
    m1 = _silu(gs[:, :H] + gd[:, :H] + d2 * wd2[...]
               + _dot(ea_ref[...], wea[...]))
    m2 = _silu(_dot(m1, we2[...]) + be2[...])
    gate = jax.nn.sigmoid(
        jnp.sum(m2 * wa[...], axis=1, keepdims=True) + ba[...])
    m = m2 * gate
    c2 = _silu(_dot(m, wc1[...]) + bc1[...])
    c = jnp.sum(c2 * wc2[...], axis=1, keepdims=True)
    row = i * EB + lax.broadcasted_iota(jnp.int32, (EB, 1), 0)
    valid = (row < ne_real).astype(F32)
    lane = lax.broadcasted_iota(jnp.int32, (EB, PW), 1)
    deg1 = (lane == 3).astype(F32)
    o_ref[:, :H] = m * valid
    o_ref[:, H:] = (rel * c + deg1) * valid


def _edge_call(ne_real, gs, gd, ea, wea, wd2, we2, be2, wa, ba, wc1, bc1, wc2):
    e_pad = gs.shape[0]
    body = functools.partial(_edge_body, ne_real)
    return pl.pallas_call(
        body,
        grid=(e_pad // EB,),
        in_specs=[pl.BlockSpec((EB, WID), lambda i: (i, 0)),
                  pl.BlockSpec((EB, WID), lambda i: (i, 0)),
                  pl.BlockSpec((EB, ED), lambda i: (i, 0)),
                  pl.BlockSpec((ED, H), lambda i: (0, 0)),
                  pl.BlockSpec((1, H), lambda i: (0, 0)),
                  pl.BlockSpec((H, H), lambda i: (0, 0)),
                  pl.BlockSpec((1, H), lambda i: (0, 0)),
                  pl.BlockSpec((1, H), lambda i: (0, 0)),
                  pl.BlockSpec((1, 1), lambda i: (0, 0)),
                  pl.BlockSpec((H, H), lambda i: (0, 0)),
                  pl.BlockSpec((1, H), lambda i: (0, 0)),
                  pl.BlockSpec((1, H), lambda i: (0, 0))],
        out_specs=pl.BlockSpec((EB, WID), lambda i: (i, 0)),
        out_shape=jax.ShapeDtypeStruct((e_pad, WID), F32),
    )(gs, gd, ea, wea, wd2, we2, be2, wa, ba, wc1, bc1, wc2)


def _node_body(h_ref, p_ref, a0_ref, a1_ref, wn1h, wn1a, bn1, wn2, bn2,
               ho_ref, po_ref):
    h = h_ref[...]
    aggm = a0_ref[:, :H] + a1_ref[:, :H]
    agg16 = a0_ref[:, H:] + a1_ref[:, H:]
    lane = lax.broadcasted_iota(jnp.int32, (NB, PW), 1)
    deg = jnp.sum(agg16 * (lane == 3).astype(F32), axis=1, keepdims=True)
    relc = jnp.where(lane < 3, agg16, 0.0)
    u = _silu(_dot(h, wn1h[...]) + _dot(aggm, wn1a[...]) + bn1[...])
    ho_ref[...] = h + _dot(u, wn2[...]) + bn2[...]
    po_ref[...] = p_ref[...] + relc / (deg + 1.0)


def _node_call(h, pos16, a0, a1, wn1h, wn1a, bn1, wn2, bn2):
    n = h.shape[0]
    return pl.pallas_call(
        _node_body,
        grid=(n // NB,),
        in_specs=[pl.BlockSpec((NB, H), lambda i: (i, 0)),
                  pl.BlockSpec((NB, PW), lambda i: (i, 0)),
                  pl.BlockSpec((NB, WID), lambda i: (i, 0)),
                  pl.BlockSpec((NB, WID), lambda i: (i, 0)),
                  pl.BlockSpec((H, H), lambda i: (0, 0)),
                  pl.BlockSpec((H, H), lambda i: (0, 0)),
                  pl.BlockSpec((1, H), lambda i: (0, 0)),
                  pl.BlockSpec((H, H), lambda i: (0, 0)),
                  pl.BlockSpec((1, H), lambda i: (0, 0))],
        out_specs=[pl.BlockSpec((NB, H), lambda i: (i, 0)),
                   pl.BlockSpec((NB, PW), lambda i: (i, 0))],
        out_shape=[jax.ShapeDtypeStruct((n, H), F32),
                   jax.ShapeDtypeStruct((n, PW), F32)],
    )(h, pos16, a0, a1, wn1h, wn1a, bn1, wn2, bn2)


def _final_body(h_ref, wo, bo, p_ref, p0_ref, t_ref, c_ref):
    t_ref[...] = _dot(h_ref[...], wo[...]) + bo[...]
    c_ref[...] = p_ref[...] - p0_ref[...]


def _final_call(h, wo, bo, pos16, pos016):
    n = h.shape[0]
    return pl.pallas_call(
        _final_body,
        grid=(n // NB,),
        in_specs=[pl.BlockSpec((NB, H), lambda i: (i, 0)),
                  pl.BlockSpec((H, NT), lambda i: (0, 0)),
                  pl.BlockSpec((1, NT), lambda i: (0, 0)),
                  pl.BlockSpec((NB, PW), lambda i: (i, 0)),
                  pl.BlockSpec((NB, PW), lambda i: (i, 0))],
        out_specs=[pl.BlockSpec((NB, NT), lambda i: (i, 0)),
                   pl.BlockSpec((NB, PW), lambda i: (i, 0))],
        out_shape=[jax.ShapeDtypeStruct((n, NT), F32),
                   jax.ShapeDtypeStruct((n, PW), F32)],
    )(h, wo, bo, pos16, pos016)


@functools.lru_cache(maxsize=None)
def _gather_kernel(n_rows, e_pad):
    chunks = e_pad // (NWORK * CH)
    per_w = chunks * CH

    @functools.partial(
        pl.kernel,
        out_type=(jax.ShapeDtypeStruct((e_pad, WID), F32),
                  jax.ShapeDtypeStruct((e_pad, WID), F32)),
        mesh=_mesh(),
        scratch_types=[pltpu.VMEM((chunks, CH), jnp.int32),
                       pltpu.VMEM((chunks, CH), jnp.int32),
                       pltpu.VMEM((2, CH, WID), F32),
                       pltpu.VMEM((2, CH, WID), F32),
                       pltpu.SemaphoreType.DMA,
                       pltpu.SemaphoreType.DMA,
                       pltpu.SemaphoreType.DMA,
                       pltpu.SemaphoreType.DMA],
        compiler_params=pltpu.CompilerParams(use_tc_tiling_on_sc=False))
    def k(tsrc, tdst, srci, dsti, gs, gd, idx_s, idx_d, row_s, row_d,
          sem_g0, sem_g1, sem_w0, sem_w1):
        w = lax.axis_index("c") * NSUB + lax.axis_index("s")
        pltpu.sync_copy(srci.at[pl.ds(w * chunks, chunks)], idx_s)
        pltpu.sync_copy(dsti.at[pl.ds(w * chunks, chunks)], idx_d)
        sem_g = [sem_g0, sem_g1]
        sem_w = [sem_w0, sem_w1]

        def body(i, carry):
            base = w * per_w + i * CH

            def run(p):
                @pl.when(i >= 2)
                def _():
                    pltpu.make_async_copy(
                        row_s.at[p], gs.at[pl.ds(base, CH)],
                        sem_w[p]).wait()
                    pltpu.make_async_copy(
                        row_d.at[p], gd.at[pl.ds(base, CH)],
                        sem_w[p]).wait()
                cs = pltpu.async_copy(tsrc.at[idx_s.at[i]], row_s.at[p],
                                      sem_g[p])
                cd = pltpu.async_copy(tdst.at[idx_d.at[i]], row_d.at[p],
                                      sem_g[p])
                cs.wait()
                cd.wait()
                pltpu.async_copy(row_s.at[p], gs.at[pl.ds(base, CH)],
                                 sem_w[p])
                pltpu.async_copy(row_d.at[p], gd.at[pl.ds(base, CH)],
                                 sem_w[p])

            @pl.when(i % 2 == 0)
            def _():
                run(0)

            @pl.when(i % 2 == 1)
            def _():
                run(1)

            return carry

        lax.fori_loop(0, chunks, body, 0)
        for p in range(2):
            pltpu.make_async_copy(row_s.at[p], gs.at[pl.ds(0, CH)],
                                  sem_w[p]).wait()
            pltpu.make_async_copy(row_d.at[p], gd.at[pl.ds(0, CH)],
                                  sem_w[p]).wait()

    return k


@functools.lru_cache(maxsize=None)
def _scatter_kernel(n_rows, e_pad):
    chunks = e_pad // (NWORK * CH)
    per_w = chunks * CH
    rpt = n_rows // NSUB
    oc = 5
    ocs = rpt // oc

    @functools.partial(
        pl.kernel,
        out_type=jax.ShapeDtypeStruct((NCORES, n_rows, WID), F32),
        mesh=_mesh(),
        scratch_types=[pltpu.VMEM_SHARED((n_rows, WID), F32),
                       pltpu.VMEM((2, CH), jnp.int32),
                       pltpu.VMEM((2, CH, WID), F32),
                       pltpu.SemaphoreType.DMA,
                       pltpu.SemaphoreType.DMA,
                       pltpu.SemaphoreType.DMA,
                       pltpu.SemaphoreType.DMA],
        compiler_params=pltpu.CompilerParams(use_tc_tiling_on_sc=False))
    def k(mr, dsti, zrows, out, acc, idx, val, sem_a0, sem_a1, sem_l0,
          sem_l1):
        cid = lax.axis_index("c")
        sid = lax.axis_index("s")
        w = cid * NSUB + sid
        pltpu.sync_copy(zrows, acc.at[pl.ds(sid * rpt, rpt)])
        plsc.subcore_barrier()
        sem_a = [sem_a0, sem_a1]
        sem_l = [sem_l0, sem_l1]

        def body(i, carry):
            base = w * per_w + i * CH

            def run(p):
                @pl.when(i >= 2)
                def _():
                    pltpu.make_async_copy(val.at[p], acc.at[idx.at[p]],
                                          sem_a[p]).wait()
                ci = pltpu.async_copy(dsti.at[w * chunks + i], idx.at[p],
                                      sem_l[p])
                cv = pltpu.async_copy(mr.at[pl.ds(base, CH)], val.at[p],
                                      sem_l[p])
                ci.wait()
                cv.wait()
                pltpu.async_copy(val.at[p], acc.at[idx.at[p]], sem_a[p],
                                 add=True)

            @pl.when(i % 2 == 0)
            def _():
                run(0)

            @pl.when(i % 2 == 1)
            def _():
                run(1)

            return carry

        lax.fori_loop(0, chunks, body, 0)
        for p in range(2):
            pltpu.make_async_copy(val.at[p], acc.at[idx.at[p]],
                                  sem_a[p]).wait()
        plsc.subcore_barrier()

        def flush(j, carry):
            s = sid * rpt + j * ocs
            pltpu.sync_copy(acc.at[pl.ds(s, ocs)], out.at[cid, pl.ds(s, ocs)])
            return carry

        lax.fori_loop(0, oc, flush, 0)

    return k


def _egnn_layer(h, pos16, src_p, dst_p, ea_p, ne_real, zrows, p):
    n = h.shape[0]
    e_pad = src_p.shape[0] * CH
    we1 = p['We1']
    tsrc, tdst = _pre_call(h, pos16, we1[:H], we1[H:2 * H],
                           p['be1'].reshape(1, H))
    gs, gd = _gather_kernel(n, e_pad)(tsrc, tdst, src_p, dst_p)
    mr = _edge_call(ne_real, gs, gd, ea_p,
                    we1[2 * H + 1:], we1[2 * H].reshape(1, H),
                    p['We2'], p['be2'].reshape(1, H),
                    p['Wa'].reshape(1, H), p['ba'].reshape(1, 1),
                    p['Wc1'], p['bc1'].reshape(1, H),
                    p['Wc2'].reshape(1, H))
    part = _scatter_kernel(n, e_pad)(mr, dst_p, zrows)
    wn1 = p['Wn1']
    return _node_call(h, pos16, part[0], part[1], wn1[:H], wn1[H:],
                      p['bn1'].reshape(1, H), p['Wn2'],
                      p['bn2'].reshape(1, H))


def _pad_edges(edge_index, edge_attr):
    e = edge_index.shape[1]
    e_pad = -(-e // (NWORK * CH)) * (NWORK * CH)
    pad = e_pad - e
    src = jnp.pad(edge_index[0].astype(jnp.int32), (0, pad)).reshape(-1, CH)
    dst = jnp.pad(edge_index[1].astype(jnp.int32), (0, pad)).reshape(-1, CH)
    ea = jnp.pad(edge_attr, ((0, pad), (0, 0)))
    return src, dst, ea, e


def kernel(lig_x, lig_pos, lig_edge_index, lig_edge_attr, pocket_x,
           pocket_pos, pocket_edge_index, pocket_edge_attr, t, lig_batch,
           pocket_batch, L_We1, L_be1, L_We2, L_be2, L_Wa, L_ba, L_Wn1,
           L_bn1, L_Wn2, L_bn2, L_Wc1, L_bc1, L_Wc2, P_We1, P_be1, P_We2,
           P_be2, P_Wa, P_ba, P_Wn1, P_bn1, P_Wn2, P_bn2, P_Wc1, P_bc1,
           P_Wc2, W_emb_l, b_emb_l, W_emb_p, b_emb_p, Wt1, bt1, Wt2, bt2,
           Wp, bp, Wo, bo):
    pnames = ['We1', 'be1', 'We2', 'be2', 'Wa', 'ba', 'Wn1', 'bn1',
              'Wn2', 'bn2', 'Wc1', 'bc1', 'Wc2']
    lw = dict(zip(pnames, [L_We1, L_be1, L_We2, L_be2, L_Wa, L_ba, L_Wn1,
                           L_bn1, L_Wn2, L_bn2, L_Wc1, L_bc1, L_Wc2]))
    pw = dict(zip(pnames, [P_We1, P_be1, P_We2, P_be2, P_Wa, P_ba, P_Wn1,
                           P_bn1, P_Wn2, P_bn2, P_Wc1, P_bc1, P_Wc2]))

    n_p = pocket_x.shape[0]
    n_l = lig_x.shape[0]
    zrows_p = jnp.zeros((n_p // NSUB, WID), F32)
    zrows_l = jnp.zeros((n_l // NSUB, WID), F32)

    hp = _emb_call(pocket_x, W_emb_p, b_emb_p.reshape(1, H))
    pp16 = jnp.pad(pocket_pos, ((0, 0), (0, PW - 3)))
    src_p, dst_p, ea_p, ne_p = _pad_edges(pocket_edge_index,
                                          pocket_edge_attr)
    for i in range(P_We1.shape[0]):
        hp, pp16 = _egnn_layer(hp, pp16, src_p, dst_p, ea_p, ne_p, zrows_p,
                               {k: v[i] for k, v in pw.items()})

    psum = _pool_call(hp, pocket_batch.astype(jnp.int32).reshape(n_p, 1))
    tcond = _temb_cond_call(t.astype(jnp.int32).reshape(NBATCH, 1),
                            Wt1, bt1.reshape(1, H), Wt2, bt2.reshape(1, H),
                            psum, Wp, bp.reshape(1, H))

    h = _emb_lig_call(lig_x, lig_batch.astype(jnp.int32).reshape(n_l, 1),
                      W_emb_l, b_emb_l.reshape(1, H), tcond)
    pl16 = jnp.pad(lig_pos, ((0, 0), (0, PW - 3)))
    src_l, dst_l, ea_l, ne_l = _pad_edges(lig_edge_index, lig_edge_attr)
    pos16 = pl16
    for i in range(L_We1.shape[0]):
        h, pos16 = _egnn_layer(h, pos16, src_l, dst_l, ea_l, ne_l, zrows_l,
                               {k: v[i] for k, v in lw.items()})

    type_pred, coord16 = _final_call(h, Wo, bo.reshape(1, NT), pos16, pl16)
    return type_pred, coord16[:, :3]


# speedup vs baseline: 1.6661x; 1.1996x over previous
import functools

import jax
import jax.numpy as jnp
from jax import lax
from jax.experimental import pallas as pl
from jax.experimental.pallas import tpu as pltpu
from jax.experimental.pallas import tpu_sc as plsc

F32 = jnp.float32
H = 128
ED = 16
NT = 10
TDIM = 128
NBATCH = 64
WID = 144
PW = 16
NB = 1000
EB = 1024
NCORES = 2
NSUB = 16
NWORK = NCORES * NSUB
CH = 128

_HI = lax.Precision.DEFAULT
BF16 = jnp.bfloat16


def _dot(a, b):
    return jnp.dot(a, b, precision=_HI)


def _silu(x):
    return x * jax.nn.sigmoid(x)


def _mesh():
    return plsc.VectorSubcoreMesh(core_axis_name="c", subcore_axis_name="s",
                                  num_cores=NCORES, num_subcores=NSUB)


def _emb_body(x_ref, w_ref, b_ref, o_ref):
    o_ref[...] = _dot(x_ref[...], w_ref[...]) + b_ref[...]


def _emb_call(x, w, b):
    n = x.shape[0]
    return pl.pallas_call(
        _emb_body,
        grid=(n // NB,),
        in_specs=[pl.BlockSpec((NB, x.shape[1]), lambda i: (i, 0)),
                  pl.BlockSpec(w.shape, lambda i: (0, 0)),
                  pl.BlockSpec(b.shape, lambda i: (0, 0))],
        out_specs=pl.BlockSpec((NB, w.shape[1]), lambda i: (i, 0)),
        out_shape=jax.ShapeDtypeStruct((n, w.shape[1]), F32),
    )(x, w, b)


def _emb_lig_body(x_ref, bt_ref, w_ref, b_ref, tc_ref, o_ref):
    oh = (bt_ref[...] == lax.broadcasted_iota(jnp.int32, (1, NBATCH), 1)
          ).astype(F32)
    o_ref[...] = (_dot(x_ref[...], w_ref[...]) + b_ref[...]
                  + _dot(oh, tc_ref[...]))


def _emb_lig_call(x, batch2d, w, b, tcond):
    n = x.shape[0]
    return pl.pallas_call(
        _emb_lig_body,
        grid=(n // NB,),
        in_specs=[pl.BlockSpec((NB, x.shape[1]), lambda i: (i, 0)),
                  pl.BlockSpec((NB, 1), lambda i: (i, 0)),
                  pl.BlockSpec(w.shape, lambda i: (0, 0)),
                  pl.BlockSpec(b.shape, lambda i: (0, 0)),
                  pl.BlockSpec(tcond.shape, lambda i: (0, 0))],
        out_specs=pl.BlockSpec((NB, H), lambda i: (i, 0)),
        out_shape=jax.ShapeDtypeStruct((n, H), F32),
    )(x, batch2d, w, b, tcond)


def _pool_body(h_ref, bt_ref, o_ref):
    i = pl.program_id(0)
    oh = (bt_ref[...] == lax.broadcasted_iota(jnp.int32, (1, NBATCH), 1)
          ).astype(F32)
    ssum = lax.dot_general(oh, h_ref[...], (((0,), (0,)), ((), ())),
                           precision=_HI)
    lane = lax.broadcasted_iota(jnp.int32, (NB, PW), 1)
    ones0 = (lane == 0).astype(F32)
    scnt = lax.dot_general(oh, ones0, (((0,), (0,)), ((), ())),
                           precision=_HI)

    @pl.when(i == 0)
    def _():
        o_ref[:, :H] = ssum
        o_ref[:, H:] = scnt

    @pl.when(i > 0)
    def _():
        o_ref[:, :H] += ssum
        o_ref[:, H:] += scnt


def _pool_call(h, batch2d):
    n = h.shape[0]
    return pl.pallas_call(
        _pool_body,
        grid=(n // NB,),
        in_specs=[pl.BlockSpec((NB, H), lambda i: (i, 0)),
                  pl.BlockSpec((NB, 1), lambda i: (i, 0))],
        out_specs=pl.BlockSpec((NBATCH, H + PW), lambda i: (0, 0)),
        out_shape=jax.ShapeDtypeStruct((NBATCH, H + PW), F32),
    )(h, batch2d)


def _temb_cond_body(t_ref, wt1, bt1, wt2, bt2, ps_ref, wp, bp, o_ref):
    t = t_ref[...].astype(F32)
    half = TDIM // 2
    k = lax.broadcasted_iota(jnp.int32, (1, half), 1).astype(F32)
    freqs = jnp.exp(-jnp.log(10000.0) * k / float(half))
    args = t * freqs
    temb = jnp.concatenate([jnp.sin(args), jnp.cos(args)], axis=1)
    temb = _silu(_dot(temb, wt1[...]) + bt1[...])
    temb = _dot(temb, wt2[...]) + bt2[...]
    ps = ps_ref[...]
    lane = lax.broadcasted_iota(jnp.int32, (NBATCH, PW), 1)
    cnt = jnp.sum(ps[:, H:] * (lane == 0).astype(F32), axis=1, keepdims=True)
    pooled = ps[:, :H] / (cnt + 1e-6)
    cond = _silu(_dot(pooled, wp[...]) + bp[...])
    o_ref[...] = temb + cond


def _temb_cond_call(t2d, wt1, bt1, wt2, bt2, psum, wp, bp):
    return pl.pallas_call(
        _temb_cond_body,
        out_shape=jax.ShapeDtypeStruct((NBATCH, H), F32),
    )(t2d, wt1, bt1, wt2, bt2, psum, wp, bp)


def _pre_body(h_ref, ws, wd, be1, ts_ref, td_ref):
    hb = h_ref[...]
    ts_ref[...] = (_dot(hb, ws[...]) + be1[...]).astype(BF16)
    td_ref[...] = _dot(hb, wd[...]).astype(BF16)


def _pre_call(h, ws, wd, be1):
    n = h.shape[0]
    return pl.pallas_call(
        _pre_body,
        grid=(n // NB,),
        in_specs=[pl.BlockSpec((NB, H), lambda i: (i, 0)),
                  pl.BlockSpec((H, H), lambda i: (0, 0)),
                  pl.BlockSpec((H, H), lambda i: (0, 0)),
                  pl.BlockSpec((1, H), lambda i: (0, 0))],
        out_specs=[pl.BlockSpec((NB, H), lambda i: (i, 0)),
                   pl.BlockSpec((NB, H), lambda i: (i, 0))],
        out_shape=[jax.ShapeDtypeStruct((n, H), BF16),
                   jax.ShapeDtypeStruct((n, H), BF16)],
    )(h, ws, wd, be1)


def _edge_body(ne_real, gs_ref, gd_ref, ps_ref, pd_ref, ea_ref, wea, wd2,
               we2, be2, wa, ba, wc1, bc1, wc2, o_ref):
    i = pl.program_id(0)
    gs = gs_ref[...].astype(F32)
    gd = gd_ref[...].astype(F32)
    rel = ps_ref[...] - pd_ref[...]
    d2 = jnp.sum(rel * rel, axis=1, keepdims=True)
    m1 = _silu(gs + gd + d2 * wd2[...]
               + _dot(ea_ref[...], wea[...]))
    m2 = _silu(_dot(m1, we2[...]) + be2[...])
    gate = jax.nn.sigmoid(
        jnp.sum(m2 * wa[...], axis=1, keepdims=True) + ba[...])
    m = m2 * gate
    c2 = _silu(_dot(m, wc1[...]) + bc1[...])
    c = jnp.sum(c2 * wc2[...], axis=1, keepdims=True)
    row = i * EB + lax.broadcasted_iota(jnp.int32, (EB, 1), 0)
    valid = (row < ne_real).astype(F32)
    lane = lax.broadcasted_iota(jnp.int32, (EB, PW), 1)
    deg1 = (lane == 3).astype(F32)
    o_ref[:, :H] = m * valid
    o_ref[:, H:] = (rel * c + deg1) * valid


def _edge_call(ne_real, gs, gd, ps, pd, ea, wea, wd2, we2, be2, wa, ba, wc1,
               bc1, wc2):
    e_pad = gs.shape[0]
    body = functools.partial(_edge_body, ne_real)
    return pl.pallas_call(
        body,
        grid=(e_pad // EB,),
        in_specs=[pl.BlockSpec((EB, H), lambda i: (i, 0)),
                  pl.BlockSpec((EB, H), lambda i: (i, 0)),
                  pl.BlockSpec((EB, PW), lambda i: (i, 0)),
                  pl.BlockSpec((EB, PW), lambda i: (i, 0)),
                  pl.BlockSpec((EB, ED), lambda i: (i, 0)),
                  pl.BlockSpec((ED, H), lambda i: (0, 0)),
                  pl.BlockSpec((1, H), lambda i: (0, 0)),
                  pl.BlockSpec((H, H), lambda i: (0, 0)),
                  pl.BlockSpec((1, H), lambda i: (0, 0)),
                  pl.BlockSpec((1, H), lambda i: (0, 0)),
                  pl.BlockSpec((1, 1), lambda i: (0, 0)),
                  pl.BlockSpec((H, H), lambda i: (0, 0)),
                  pl.BlockSpec((1, H), lambda i: (0, 0)),
                  pl.BlockSpec((1, H), lambda i: (0, 0))],
        out_specs=pl.BlockSpec((EB, WID), lambda i: (i, 0)),
        out_shape=jax.ShapeDtypeStruct((e_pad, WID), F32),
    )(gs, gd, ps, pd, ea, wea, wd2, we2, be2, wa, ba, wc1, bc1, wc2)


def _node_body(h_ref, p_ref, a0_ref, a1_ref, wn1h, wn1a, bn1, wn2, bn2,
               ho_ref, po_ref):
    h = h_ref[...]
    aggm = a0_ref[:, :H] + a1_ref[:, :H]
    agg16 = a0_ref[:, H:] + a1_ref[:, H:]
    lane = lax.broadcasted_iota(jnp.int32, (NB, PW), 1)
    deg = jnp.sum(agg16 * (lane == 3).astype(F32), axis=1, keepdims=True)
    relc = jnp.where(lane < 3, agg16, 0.0)
    u = _silu(_dot(h, wn1h[...]) + _dot(aggm, wn1a[...]) + bn1[...])
    ho_ref[...] = h + _dot(u, wn2[...]) + bn2[...]
    po_ref[...] = p_ref[...] + relc / (deg + 1.0)


def _node_call(h, pos16, a0, a1, wn1h, wn1a, bn1, wn2, bn2):
    n = h.shape[0]
    return pl.pallas_call(
        _node_body,
        grid=(n // NB,),
        in_specs=[pl.BlockSpec((NB, H), lambda i: (i, 0)),
                  pl.BlockSpec((NB, PW), lambda i: (i, 0)),
                  pl.BlockSpec((NB, WID), lambda i: (i, 0)),
                  pl.BlockSpec((NB, WID), lambda i: (i, 0)),
                  pl.BlockSpec((H, H), lambda i: (0, 0)),
                  pl.BlockSpec((H, H), lambda i: (0, 0)),
                  pl.BlockSpec((1, H), lambda i: (0, 0)),
                  pl.BlockSpec((H, H), lambda i: (0, 0)),
                  pl.BlockSpec((1, H), lambda i: (0, 0))],
        out_specs=[pl.BlockSpec((NB, H), lambda i: (i, 0)),
                   pl.BlockSpec((NB, PW), lambda i: (i, 0))],
        out_shape=[jax.ShapeDtypeStruct((n, H), F32),
                   jax.ShapeDtypeStruct((n, PW), F32)],
    )(h, pos16, a0, a1, wn1h, wn1a, bn1, wn2, bn2)


def _final_body(h_ref, wo, bo, p_ref, p0_ref, t_ref, c_ref):
    t_ref[...] = _dot(h_ref[...], wo[...]) + bo[...]
    c_ref[...] = p_ref[...] - p0_ref[...]


def _final_call(h, wo, bo, pos16, pos016):
    n = h.shape[0]
    return pl.pallas_call(
        _final_body,
        grid=(n // NB,),
        in_specs=[pl.BlockSpec((NB, H), lambda i: (i, 0)),
                  pl.BlockSpec((H, NT), lambda i: (0, 0)),
                  pl.BlockSpec((1, NT), lambda i: (0, 0)),
                  pl.BlockSpec((NB, PW), lambda i: (i, 0)),
                  pl.BlockSpec((NB, PW), lambda i: (i, 0))],
        out_specs=[pl.BlockSpec((NB, NT), lambda i: (i, 0)),
                   pl.BlockSpec((NB, PW), lambda i: (i, 0))],
        out_shape=[jax.ShapeDtypeStruct((n, NT), F32),
                   jax.ShapeDtypeStruct((n, PW), F32)],
    )(h, wo, bo, pos16, pos016)


@functools.lru_cache(maxsize=None)
def _gather_kernel(n_rows, e_pad):
    chunks = e_pad // (NWORK * CH)
    per_w = chunks * CH

    @functools.partial(
        pl.kernel,
        out_type=(jax.ShapeDtypeStruct((e_pad, H), BF16),
                  jax.ShapeDtypeStruct((e_pad, H), BF16),
                  jax.ShapeDtypeStruct((e_pad, PW), F32),
                  jax.ShapeDtypeStruct((e_pad, PW), F32)),
        mesh=_mesh(),
        scratch_types=[pltpu.VMEM((chunks, CH), jnp.int32),
                       pltpu.VMEM((chunks, CH), jnp.int32),
                       pltpu.VMEM((2, CH, H), BF16),
                       pltpu.VMEM((2, CH, H), BF16),
                       pltpu.VMEM((2, CH, PW), F32),
                       pltpu.VMEM((2, CH, PW), F32),
                       pltpu.SemaphoreType.DMA,
                       pltpu.SemaphoreType.DMA,
                       pltpu.SemaphoreType.DMA,
                       pltpu.SemaphoreType.DMA],
        compiler_params=pltpu.CompilerParams(use_tc_tiling_on_sc=False))
    def k(tsrc, tdst, pos, srci, dsti, gs, gd, qs, qd, idx_s_ref, idx_d_ref,
          row_s, row_d, pr_s, pr_d, sem_g0, sem_g1, sem_w0, sem_w1):
        w = lax.axis_index("c") * NSUB + lax.axis_index("s")
        pltpu.sync_copy(srci.at[pl.ds(w * chunks, chunks)], idx_s_ref)
        pltpu.sync_copy(dsti.at[pl.ds(w * chunks, chunks)], idx_d_ref)
        sem_g = [sem_g0, sem_g1]
        sem_w = [sem_w0, sem_w1]

        def body(i, carry):
            base = w * per_w + i * CH

            def run(p):
                @pl.when(i >= 2)
                def _():
                    pltpu.make_async_copy(
                        row_s.at[p], gs.at[pl.ds(base, CH)],
                        sem_w[p]).wait()
                    pltpu.make_async_copy(
                        row_d.at[p], gd.at[pl.ds(base, CH)],
                        sem_w[p]).wait()
                    pltpu.make_async_copy(
                        pr_s.at[p], qs.at[pl.ds(base, CH)],
                        sem_w[p]).wait()
                    pltpu.make_async_copy(
                        pr_d.at[p], qd.at[pl.ds(base, CH)],
                        sem_w[p]).wait()
                cps = [
                    pltpu.async_copy(tsrc.at[idx_s_ref.at[i]], row_s.at[p],
                                     sem_g[p]),
                    pltpu.async_copy(tdst.at[idx_d_ref.at[i]], row_d.at[p],
                                     sem_g[p]),
                    pltpu.async_copy(pos.at[idx_s_ref.at[i]], pr_s.at[p],
                                     sem_g[p]),
                    pltpu.async_copy(pos.at[idx_d_ref.at[i]], pr_d.at[p],
                                     sem_g[p]),
                ]
                for c in cps:
                    c.wait()
                pltpu.async_copy(row_s.at[p], gs.at[pl.ds(base, CH)],
                                 sem_w[p])
                pltpu.async_copy(row_d.at[p], gd.at[pl.ds(base, CH)],
                                 sem_w[p])
                pltpu.async_copy(pr_s.at[p], qs.at[pl.ds(base, CH)],
                                 sem_w[p])
                pltpu.async_copy(pr_d.at[p], qd.at[pl.ds(base, CH)],
                                 sem_w[p])

            @pl.when(i % 2 == 0)
            def _():
                run(0)

            @pl.when(i % 2 == 1)
            def _():
                run(1)

            return carry

        lax.fori_loop(0, chunks, body, 0)
        for p in range(2):
            pltpu.make_async_copy(row_s.at[p], gs.at[pl.ds(0, CH)],
                                  sem_w[p]).wait()
            pltpu.make_async_copy(row_d.at[p], gd.at[pl.ds(0, CH)],
                                  sem_w[p]).wait()
            pltpu.make_async_copy(pr_s.at[p], qs.at[pl.ds(0, CH)],
                                  sem_w[p]).wait()
            pltpu.make_async_copy(pr_d.at[p], qd.at[pl.ds(0, CH)],
                                  sem_w[p]).wait()

    return k


@functools.lru_cache(maxsize=None)
def _scatter_kernel(n_rows, e_pad):
    chunks = e_pad // (NWORK * CH)
    per_w = chunks * CH
    rpt = n_rows // NSUB
    oc = 5
    ocs = rpt // oc

    @functools.partial(
        pl.kernel,
        out_type=jax.ShapeDtypeStruct((NCORES, n_rows, WID), F32),
        mesh=_mesh(),
        scratch_types=[pltpu.VMEM_SHARED((n_rows, WID), F32),
                       pltpu.VMEM((2, CH), jnp.int32),
                       pltpu.VMEM((2, CH, WID), F32),
                       pltpu.SemaphoreType.DMA,
                       pltpu.SemaphoreType.DMA,
                       pltpu.SemaphoreType.DMA,
                       pltpu.SemaphoreType.DMA],
        compiler_params=pltpu.CompilerParams(use_tc_tiling_on_sc=False))
    def k(mr, dsti, zrows, out, acc, idx, val, sem_a0, sem_a1, sem_l0,
          sem_l1):
        cid = lax.axis_index("c")
        sid = lax.axis_index("s")
        w = cid * NSUB + sid
        pltpu.sync_copy(zrows, acc.at[pl.ds(sid * rpt, rpt)])
        plsc.subcore_barrier()
        sem_a = [sem_a0, sem_a1]
        sem_l = [sem_l0, sem_l1]

        def body(i, carry):
            base = w * per_w + i * CH

            def run(p):
                @pl.when(i >= 2)
                def _():
                    pltpu.make_async_copy(val.at[p], acc.at[idx.at[p]],
                                          sem_a[p]).wait()
                ci = pltpu.async_copy(dsti.at[w * chunks + i], idx.at[p],
                                      sem_l[p])
                cv = pltpu.async_copy(mr.at[pl.ds(base, CH)], val.at[p],
                                      sem_l[p])
                ci.wait()
                cv.wait()
                pltpu.async_copy(val.at[p], acc.at[idx.at[p]], sem_a[p],
                                 add=True)

            @pl.when(i % 2 == 0)
            def _():
                run(0)

            @pl.when(i % 2 == 1)
            def _():
                run(1)

            return carry

        lax.fori_loop(0, chunks, body, 0)
        for p in range(2):
            pltpu.make_async_copy(val.at[p], acc.at[idx.at[p]],
                                  sem_a[p]).wait()
        plsc.subcore_barrier()

        def flush(j, carry):
            s = sid * rpt + j * ocs
            pltpu.sync_copy(acc.at[pl.ds(s, ocs)], out.at[cid, pl.ds(s, ocs)])
            return carry

        lax.fori_loop(0, oc, flush, 0)

    return k


def _egnn_layer(h, pos16, src_p, dst_p, ea_p, ne_real, zrows, p):
    n = h.shape[0]
    e_pad = src_p.shape[0] * CH
    we1 = p['We1']
    tsrc, tdst = _pre_call(h, we1[:H], we1[H:2 * H], p['be1'].reshape(1, H))
    gs, gd, qs, qd = _gather_kernel(n, e_pad)(tsrc, tdst, pos16, src_p,
                                              dst_p)
    mr = _edge_call(ne_real, gs, gd, qs, qd, ea_p,
                    we1[2 * H + 1:], we1[2 * H].reshape(1, H),
                    p['We2'], p['be2'].reshape(1, H),
                    p['Wa'].reshape(1, H), p['ba'].reshape(1, 1),
                    p['Wc1'], p['bc1'].reshape(1, H),
                    p['Wc2'].reshape(1, H))
    part = _scatter_kernel(n, e_pad)(mr, dst_p, zrows)
    wn1 = p['Wn1']
    return _node_call(h, pos16, part[0], part[1], wn1[:H], wn1[H:],
                      p['bn1'].reshape(1, H), p['Wn2'],
                      p['bn2'].reshape(1, H))


def _pad_edges(edge_index, edge_attr):
    e = edge_index.shape[1]
    e_pad = -(-e // (NWORK * CH)) * (NWORK * CH)
    pad = e_pad - e
    src = jnp.pad(edge_index[0].astype(jnp.int32), (0, pad)).reshape(-1, CH)
    dst = jnp.pad(edge_index[1].astype(jnp.int32), (0, pad)).reshape(-1, CH)
    ea = jnp.pad(edge_attr, ((0, pad), (0, 0)))
    return src, dst, ea, e


def kernel(lig_x, lig_pos, lig_edge_index, lig_edge_attr, pocket_x,
           pocket_pos, pocket_edge_index, pocket_edge_attr, t, lig_batch,
           pocket_batch, L_We1, L_be1, L_We2, L_be2, L_Wa, L_ba, L_Wn1,
           L_bn1, L_Wn2, L_bn2, L_Wc1, L_bc1, L_Wc2, P_We1, P_be1, P_We2,
           P_be2, P_Wa, P_ba, P_Wn1, P_bn1, P_Wn2, P_bn2, P_Wc1, P_bc1,
           P_Wc2, W_emb_l, b_emb_l, W_emb_p, b_emb_p, Wt1, bt1, Wt2, bt2,
           Wp, bp, Wo, bo):
    pnames = ['We1', 'be1', 'We2', 'be2', 'Wa', 'ba', 'Wn1', 'bn1',
              'Wn2', 'bn2', 'Wc1', 'bc1', 'Wc2']
    lw = dict(zip(pnames, [L_We1, L_be1, L_We2, L_be2, L_Wa, L_ba, L_Wn1,
                           L_bn1, L_Wn2, L_bn2, L_Wc1, L_bc1, L_Wc2]))
    pw = dict(zip(pnames, [P_We1, P_be1, P_We2, P_be2, P_Wa, P_ba, P_Wn1,
                           P_bn1, P_Wn2, P_bn2, P_Wc1, P_bc1, P_Wc2]))

    n_p = pocket_x.shape[0]
    n_l = lig_x.shape[0]
    zrows_p = jnp.zeros((n_p // NSUB, WID), F32)
    zrows_l = jnp.zeros((n_l // NSUB, WID), F32)

    hp = _emb_call(pocket_x, W_emb_p, b_emb_p.reshape(1, H))
    pp16 = jnp.pad(pocket_pos, ((0, 0), (0, PW - 3)))
    src_p, dst_p, ea_p, ne_p = _pad_edges(pocket_edge_index,
                                          pocket_edge_attr)
    for i in range(P_We1.shape[0]):
        hp, pp16 = _egnn_layer(hp, pp16, src_p, dst_p, ea_p, ne_p, zrows_p,
                               {k: v[i] for k, v in pw.items()})

    psum = _pool_call(hp, pocket_batch.astype(jnp.int32).reshape(n_p, 1))
    tcond = _temb_cond_call(t.astype(jnp.int32).reshape(NBATCH, 1),
                            Wt1, bt1.reshape(1, H), Wt2, bt2.reshape(1, H),
                            psum, Wp, bp.reshape(1, H))

    h = _emb_lig_call(lig_x, lig_batch.astype(jnp.int32).reshape(n_l, 1),
                      W_emb_l, b_emb_l.reshape(1, H), tcond)
    pl16 = jnp.pad(lig_pos, ((0, 0), (0, PW - 3)))
    src_l, dst_l, ea_l, ne_l = _pad_edges(lig_edge_index, lig_edge_attr)
    pos16 = pl16
    for i in range(L_We1.shape[0]):
        h, pos16 = _egnn_layer(h, pos16, src_l, dst_l, ea_l, ne_l, zrows_l,
                               {k: v[i] for k, v in lw.items()})

    type_pred, coord16 = _final_call(h, Wo, bo.reshape(1, NT), pos16, pl16)
    return type_pred, coord16[:, :3]


# speedup vs baseline: 2.1311x; 1.2791x over previous
import functools

import jax
import jax.numpy as jnp
from jax import lax
from jax.experimental import pallas as pl
from jax.experimental.pallas import tpu as pltpu
from jax.experimental.pallas import tpu_sc as plsc

F32 = jnp.float32
H = 128
ED = 16
NT = 10
TDIM = 128
NBATCH = 64
WID = 144
PW = 16
NB = 1000
EB = 1024
NCORES = 2
NSUB = 16
NWORK = NCORES * NSUB
CH = 128

_HI = lax.Precision.DEFAULT
BF16 = jnp.bfloat16


def _dot(a, b):
    return jnp.dot(a, b, precision=_HI)


def _silu(x):
    return x * jax.nn.sigmoid(x)


def _mesh():
    return plsc.VectorSubcoreMesh(core_axis_name="c", subcore_axis_name="s",
                                  num_cores=NCORES, num_subcores=NSUB)


def _emb_body(x_ref, w_ref, b_ref, o_ref):
    o_ref[...] = _dot(x_ref[...], w_ref[...]) + b_ref[...]


def _emb_call(x, w, b):
    n = x.shape[0]
    return pl.pallas_call(
        _emb_body,
        grid=(n // NB,),
        in_specs=[pl.BlockSpec((NB, x.shape[1]), lambda i: (i, 0)),
                  pl.BlockSpec(w.shape, lambda i: (0, 0)),
                  pl.BlockSpec(b.shape, lambda i: (0, 0))],
        out_specs=pl.BlockSpec((NB, w.shape[1]), lambda i: (i, 0)),
        out_shape=jax.ShapeDtypeStruct((n, w.shape[1]), F32),
    )(x, w, b)


def _emb_lig_body(x_ref, bt_ref, w_ref, b_ref, tc_ref, o_ref):
    oh = (bt_ref[...] == lax.broadcasted_iota(jnp.int32, (1, NBATCH), 1)
          ).astype(F32)
    o_ref[...] = (_dot(x_ref[...], w_ref[...]) + b_ref[...]
                  + _dot(oh, tc_ref[...]))


def _emb_lig_call(x, batch2d, w, b, tcond):
    n = x.shape[0]
    return pl.pallas_call(
        _emb_lig_body,
        grid=(n // NB,),
        in_specs=[pl.BlockSpec((NB, x.shape[1]), lambda i: (i, 0)),
                  pl.BlockSpec((NB, 1), lambda i: (i, 0)),
                  pl.BlockSpec(w.shape, lambda i: (0, 0)),
                  pl.BlockSpec(b.shape, lambda i: (0, 0)),
                  pl.BlockSpec(tcond.shape, lambda i: (0, 0))],
        out_specs=pl.BlockSpec((NB, H), lambda i: (i, 0)),
        out_shape=jax.ShapeDtypeStruct((n, H), F32),
    )(x, batch2d, w, b, tcond)


def _pool_body(h_ref, bt_ref, o_ref):
    i = pl.program_id(0)
    oh = (bt_ref[...] == lax.broadcasted_iota(jnp.int32, (1, NBATCH), 1)
          ).astype(F32)
    ssum = lax.dot_general(oh, h_ref[...], (((0,), (0,)), ((), ())),
                           precision=_HI)
    lane = lax.broadcasted_iota(jnp.int32, (NB, PW), 1)
    ones0 = (lane == 0).astype(F32)
    scnt = lax.dot_general(oh, ones0, (((0,), (0,)), ((), ())),
                           precision=_HI)

    @pl.when(i == 0)
    def _():
        o_ref[:, :H] = ssum
        o_ref[:, H:] = scnt

    @pl.when(i > 0)
    def _():
        o_ref[:, :H] += ssum
        o_ref[:, H:] += scnt


def _pool_call(h, batch2d):
    n = h.shape[0]
    return pl.pallas_call(
        _pool_body,
        grid=(n // NB,),
        in_specs=[pl.BlockSpec((NB, H), lambda i: (i, 0)),
                  pl.BlockSpec((NB, 1), lambda i: (i, 0))],
        out_specs=pl.BlockSpec((NBATCH, H + PW), lambda i: (0, 0)),
        out_shape=jax.ShapeDtypeStruct((NBATCH, H + PW), F32),
    )(h, batch2d)


def _temb_cond_body(t_ref, wt1, bt1, wt2, bt2, ps_ref, wp, bp, o_ref):
    t = t_ref[...].astype(F32)
    half = TDIM // 2
    k = lax.broadcasted_iota(jnp.int32, (1, half), 1).astype(F32)
    freqs = jnp.exp(-jnp.log(10000.0) * k / float(half))
    args = t * freqs
    temb = jnp.concatenate([jnp.sin(args), jnp.cos(args)], axis=1)
    temb = _silu(_dot(temb, wt1[...]) + bt1[...])
    temb = _dot(temb, wt2[...]) + bt2[...]
    ps = ps_ref[...]
    lane = lax.broadcasted_iota(jnp.int32, (NBATCH, PW), 1)
    cnt = jnp.sum(ps[:, H:] * (lane == 0).astype(F32), axis=1, keepdims=True)
    pooled = ps[:, :H] / (cnt + 1e-6)
    cond = _silu(_dot(pooled, wp[...]) + bp[...])
    o_ref[...] = temb + cond


def _temb_cond_call(t2d, wt1, bt1, wt2, bt2, psum, wp, bp):
    return pl.pallas_call(
        _temb_cond_body,
        out_shape=jax.ShapeDtypeStruct((NBATCH, H), F32),
    )(t2d, wt1, bt1, wt2, bt2, psum, wp, bp)


def _pre_body(h_ref, ws, wd, be1, ts_ref, td_ref):
    hb = h_ref[...]
    ts_ref[...] = _dot(hb, ws[...]) + be1[...]
    td_ref[...] = _dot(hb, wd[...])


def _pre_call(h, ws, wd, be1):
    n = h.shape[0]
    return pl.pallas_call(
        _pre_body,
        grid=(n // NB,),
        in_specs=[pl.BlockSpec((NB, H), lambda i: (i, 0)),
                  pl.BlockSpec((H, H), lambda i: (0, 0)),
                  pl.BlockSpec((H, H), lambda i: (0, 0)),
                  pl.BlockSpec((1, H), lambda i: (0, 0))],
        out_specs=[pl.BlockSpec((NB, H), lambda i: (i, 0)),
                   pl.BlockSpec((NB, H), lambda i: (i, 0))],
        out_shape=[jax.ShapeDtypeStruct((n, H), F32),
                   jax.ShapeDtypeStruct((n, H), F32)],
    )(h, ws, wd, be1)


def _edge_body(ne_real, gs_ref, gd_ref, ps_ref, pd_ref, ea_ref, wea, wd2,
               we2, be2, wa, ba, wc1, bc1, wc2, om_ref, or_ref):
    i = pl.program_id(0)
    gs = gs_ref[...]
    gd = gd_ref[...]
    rel = ps_ref[...] - pd_ref[...]
    d2 = jnp.sum(rel * rel, axis=1, keepdims=True)
    m1 = _silu(gs + gd + d2 * wd2[...]
               + _dot(ea_ref[...], wea[...]))
    m2 = _silu(_dot(m1, we2[...]) + be2[...])
    gate = jax.nn.sigmoid(
        jnp.sum(m2 * wa[...], axis=1, keepdims=True) + ba[...])
    m = m2 * gate
    c2 = _silu(_dot(m, wc1[...]) + bc1[...])
    c = jnp.sum(c2 * wc2[...], axis=1, keepdims=True)
    row = i * EB + lax.broadcasted_iota(jnp.int32, (EB, 1), 0)
    valid = (row < ne_real).astype(F32)
    lane = lax.broadcasted_iota(jnp.int32, (EB, PW), 1)
    deg1 = (lane == 3).astype(F32)
    om_ref[...] = m * valid
    or_ref[...] = (rel * c + deg1) * valid


def _edge_call(ne_real, gs, gd, ps, pd, ea, wea, wd2, we2, be2, wa, ba, wc1,
               bc1, wc2):
    e_pad = gs.shape[0]
    nblk = e_pad // EB
    last = ne_real // EB - (1 if ne_real % EB == 0 else 0)
    body = functools.partial(_edge_body, ne_real)

    def ea_map(i):
        return (jnp.minimum(i, last), 0)

    return pl.pallas_call(
        body,
        grid=(nblk,),
        in_specs=[pl.BlockSpec((EB, H), lambda i: (i, 0)),
                  pl.BlockSpec((EB, H), lambda i: (i, 0)),
                  pl.BlockSpec((EB, PW), lambda i: (i, 0)),
                  pl.BlockSpec((EB, PW), lambda i: (i, 0)),
                  pl.BlockSpec((EB, ED), ea_map),
                  pl.BlockSpec((ED, H), lambda i: (0, 0)),
                  pl.BlockSpec((1, H), lambda i: (0, 0)),
                  pl.BlockSpec((H, H), lambda i: (0, 0)),
                  pl.BlockSpec((1, H), lambda i: (0, 0)),
                  pl.BlockSpec((1, H), lambda i: (0, 0)),
                  pl.BlockSpec((1, 1), lambda i: (0, 0)),
                  pl.BlockSpec((H, H), lambda i: (0, 0)),
                  pl.BlockSpec((1, H), lambda i: (0, 0)),
                  pl.BlockSpec((1, H), lambda i: (0, 0))],
        out_specs=[pl.BlockSpec((EB, H), lambda i: (i, 0)),
                   pl.BlockSpec((EB, PW), lambda i: (i, 0))],
        out_shape=[jax.ShapeDtypeStruct((e_pad, H), F32),
                   jax.ShapeDtypeStruct((e_pad, PW), F32)],
    )(gs, gd, ps, pd, ea, wea, wd2, we2, be2, wa, ba, wc1, bc1, wc2)


def _node_body(h_ref, p_ref, a0m_ref, a1m_ref, a0r_ref, a1r_ref, wn1h, wn1a,
               bn1, wn2, bn2, ho_ref, po_ref):
    h = h_ref[...]
    aggm = a0m_ref[...] + a1m_ref[...]
    agg16 = a0r_ref[...] + a1r_ref[...]
    lane = lax.broadcasted_iota(jnp.int32, (NB, PW), 1)
    deg = jnp.sum(agg16 * (lane == 3).astype(F32), axis=1, keepdims=True)
    relc = jnp.where(lane < 3, agg16, 0.0)
    u = _silu(_dot(h, wn1h[...]) + _dot(aggm, wn1a[...]) + bn1[...])
    ho_ref[...] = h + _dot(u, wn2[...]) + bn2[...]
    po_ref[...] = p_ref[...] + relc / (deg + 1.0)


def _node_call(h, pos16, a0m, a1m, a0r, a1r, wn1h, wn1a, bn1, wn2, bn2):
    n = h.shape[0]
    return pl.pallas_call(
        _node_body,
        grid=(n // NB,),
        in_specs=[pl.BlockSpec((NB, H), lambda i: (i, 0)),
                  pl.BlockSpec((NB, PW), lambda i: (i, 0)),
                  pl.BlockSpec((NB, H), lambda i: (i, 0)),
                  pl.BlockSpec((NB, H), lambda i: (i, 0)),
                  pl.BlockSpec((NB, PW), lambda i: (i, 0)),
                  pl.BlockSpec((NB, PW), lambda i: (i, 0)),
                  pl.BlockSpec((H, H), lambda i: (0, 0)),
                  pl.BlockSpec((H, H), lambda i: (0, 0)),
                  pl.BlockSpec((1, H), lambda i: (0, 0)),
                  pl.BlockSpec((H, H), lambda i: (0, 0)),
                  pl.BlockSpec((1, H), lambda i: (0, 0))],
        out_specs=[pl.BlockSpec((NB, H), lambda i: (i, 0)),
                   pl.BlockSpec((NB, PW), lambda i: (i, 0))],
        out_shape=[jax.ShapeDtypeStruct((n, H), F32),
                   jax.ShapeDtypeStruct((n, PW), F32)],
    )(h, pos16, a0m, a1m, a0r, a1r, wn1h, wn1a, bn1, wn2, bn2)


def _final_body(h_ref, wo, bo, p_ref, p0_ref, t_ref, c_ref):
    t_ref[...] = _dot(h_ref[...], wo[...]) + bo[...]
    c_ref[...] = p_ref[...] - p0_ref[...]


def _final_call(h, wo, bo, pos16, pos016):
    n = h.shape[0]
    return pl.pallas_call(
        _final_body,
        grid=(n // NB,),
        in_specs=[pl.BlockSpec((NB, H), lambda i: (i, 0)),
                  pl.BlockSpec((H, NT), lambda i: (0, 0)),
                  pl.BlockSpec((1, NT), lambda i: (0, 0)),
                  pl.BlockSpec((NB, PW), lambda i: (i, 0)),
                  pl.BlockSpec((NB, PW), lambda i: (i, 0))],
        out_specs=[pl.BlockSpec((NB, NT), lambda i: (i, 0)),
                   pl.BlockSpec((NB, PW), lambda i: (i, 0))],
        out_shape=[jax.ShapeDtypeStruct((n, NT), F32),
                   jax.ShapeDtypeStruct((n, PW), F32)],
    )(h, wo, bo, pos16, pos016)


@functools.lru_cache(maxsize=None)
def _gather_kernel(n_rows, e_pad):
    chunks = e_pad // (NWORK * CH)
    per_w = chunks * CH

    @functools.partial(
        pl.kernel,
        out_type=(jax.ShapeDtypeStruct((e_pad, H), F32),
                  jax.ShapeDtypeStruct((e_pad, H), F32),
                  jax.ShapeDtypeStruct((e_pad, PW), F32),
                  jax.ShapeDtypeStruct((e_pad, PW), F32)),
        mesh=_mesh(),
        scratch_types=[pltpu.VMEM((chunks, CH), jnp.int32),
                       pltpu.VMEM((chunks, CH), jnp.int32),
                       pltpu.VMEM((2, CH, H), F32),
                       pltpu.VMEM((2, CH, H), F32),
                       pltpu.VMEM((2, CH, PW), F32),
                       pltpu.VMEM((2, CH, PW), F32),
                       pltpu.SemaphoreType.DMA,
                       pltpu.SemaphoreType.DMA,
                       pltpu.SemaphoreType.DMA,
                       pltpu.SemaphoreType.DMA],
        compiler_params=pltpu.CompilerParams(use_tc_tiling_on_sc=False))
    def k(tsrc, tdst, pos, srci, dsti, gs, gd, qs, qd, idx_s_ref, idx_d_ref,
          row_s, row_d, pr_s, pr_d, sem_g0, sem_g1, sem_w0, sem_w1):
        w = lax.axis_index("c") * NSUB + lax.axis_index("s")
        pltpu.sync_copy(srci.at[pl.ds(w * chunks, chunks)], idx_s_ref)
        pltpu.sync_copy(dsti.at[pl.ds(w * chunks, chunks)], idx_d_ref)
        sem_g = [sem_g0, sem_g1]
        sem_w = [sem_w0, sem_w1]

        def body(i, carry):
            base = w * per_w + i * CH

            def run(p):
                @pl.when(i >= 2)
                def _():
                    pltpu.make_async_copy(
                        row_s.at[p], gs.at[pl.ds(base, CH)],
                        sem_w[p]).wait()
                    pltpu.make_async_copy(
                        row_d.at[p], gd.at[pl.ds(base, CH)],
                        sem_w[p]).wait()
                    pltpu.make_async_copy(
                        pr_s.at[p], qs.at[pl.ds(base, CH)],
                        sem_w[p]).wait()
                    pltpu.make_async_copy(
                        pr_d.at[p], qd.at[pl.ds(base, CH)],
                        sem_w[p]).wait()
                cps = [
                    pltpu.async_copy(tsrc.at[idx_s_ref.at[i]], row_s.at[p],
                                     sem_g[p]),
                    pltpu.async_copy(tdst.at[idx_d_ref.at[i]], row_d.at[p],
                                     sem_g[p]),
                    pltpu.async_copy(pos.at[idx_s_ref.at[i]], pr_s.at[p],
                                     sem_g[p]),
                    pltpu.async_copy(pos.at[idx_d_ref.at[i]], pr_d.at[p],
                                     sem_g[p]),
                ]
                for c in cps:
                    c.wait()
                pltpu.async_copy(row_s.at[p], gs.at[pl.ds(base, CH)],
                                 sem_w[p])
                pltpu.async_copy(row_d.at[p], gd.at[pl.ds(base, CH)],
                                 sem_w[p])
                pltpu.async_copy(pr_s.at[p], qs.at[pl.ds(base, CH)],
                                 sem_w[p])
                pltpu.async_copy(pr_d.at[p], qd.at[pl.ds(base, CH)],
                                 sem_w[p])

            @pl.when(i % 2 == 0)
            def _():
                run(0)

            @pl.when(i % 2 == 1)
            def _():
                run(1)

            return carry

        lax.fori_loop(0, chunks, body, 0)
        for p in range(2):
            pltpu.make_async_copy(row_s.at[p], gs.at[pl.ds(0, CH)],
                                  sem_w[p]).wait()
            pltpu.make_async_copy(row_d.at[p], gd.at[pl.ds(0, CH)],
                                  sem_w[p]).wait()
            pltpu.make_async_copy(pr_s.at[p], qs.at[pl.ds(0, CH)],
                                  sem_w[p]).wait()
            pltpu.make_async_copy(pr_d.at[p], qd.at[pl.ds(0, CH)],
                                  sem_w[p]).wait()

    return k


@functools.lru_cache(maxsize=None)
def _scatter_kernel(n_rows, e_pad):
    chunks = e_pad // (NWORK * CH)
    per_w = chunks * CH
    rpt = n_rows // NSUB
    oc = 5
    ocs = rpt // oc

    @functools.partial(
        pl.kernel,
        out_type=(jax.ShapeDtypeStruct((NCORES, n_rows, H), F32),
                  jax.ShapeDtypeStruct((NCORES, n_rows, PW), F32)),
        mesh=_mesh(),
        scratch_types=[pltpu.VMEM_SHARED((n_rows, H), F32),
                       pltpu.VMEM_SHARED((n_rows, PW), F32),
                       pltpu.VMEM((2, CH), jnp.int32),
                       pltpu.VMEM((2, CH, H), F32),
                       pltpu.VMEM((2, CH, PW), F32),
                       pltpu.SemaphoreType.DMA,
                       pltpu.SemaphoreType.DMA,
                       pltpu.SemaphoreType.DMA,
                       pltpu.SemaphoreType.DMA],
        compiler_params=pltpu.CompilerParams(use_tc_tiling_on_sc=False))
    def k(mrm, mrr, dsti, zm, zr, outm, outr, accm, accr, idx, valm, valr,
          sem_a0, sem_a1, sem_l0, sem_l1):
        cid = lax.axis_index("c")
        sid = lax.axis_index("s")
        w = cid * NSUB + sid
        pltpu.sync_copy(zm, accm.at[pl.ds(sid * rpt, rpt)])
        pltpu.sync_copy(zr, accr.at[pl.ds(sid * rpt, rpt)])
        plsc.subcore_barrier()
        sem_a = [sem_a0, sem_a1]
        sem_l = [sem_l0, sem_l1]

        def body(i, carry):
            base = w * per_w + i * CH

            def run(p):
                @pl.when(i >= 2)
                def _():
                    pltpu.make_async_copy(valm.at[p], accm.at[idx.at[p]],
                                          sem_a[p]).wait()
                    pltpu.make_async_copy(valr.at[p], accr.at[idx.at[p]],
                                          sem_a[p]).wait()
                cps = [
                    pltpu.async_copy(dsti.at[w * chunks + i], idx.at[p],
                                     sem_l[p]),
                    pltpu.async_copy(mrm.at[pl.ds(base, CH)], valm.at[p],
                                     sem_l[p]),
                    pltpu.async_copy(mrr.at[pl.ds(base, CH)], valr.at[p],
                                     sem_l[p]),
                ]
                for c in cps:
                    c.wait()
                pltpu.async_copy(valm.at[p], accm.at[idx.at[p]], sem_a[p],
                                 add=True)
                pltpu.async_copy(valr.at[p], accr.at[idx.at[p]], sem_a[p],
                                 add=True)

            @pl.when(i % 2 == 0)
            def _():
                run(0)

            @pl.when(i % 2 == 1)
            def _():
                run(1)

            return carry

        lax.fori_loop(0, chunks, body, 0)
        for p in range(2):
            pltpu.make_async_copy(valm.at[p], accm.at[idx.at[p]],
                                  sem_a[p]).wait()
            pltpu.make_async_copy(valr.at[p], accr.at[idx.at[p]],
                                  sem_a[p]).wait()
        plsc.subcore_barrier()

        def flush(j, carry):
            s = sid * rpt + j * ocs
            pltpu.sync_copy(accm.at[pl.ds(s, ocs)],
                            outm.at[cid, pl.ds(s, ocs)])
            pltpu.sync_copy(accr.at[pl.ds(s, ocs)],
                            outr.at[cid, pl.ds(s, ocs)])
            return carry

        lax.fori_loop(0, oc, flush, 0)

    return k


def _egnn_layer(h, pos16, src_p, dst_p, ea, ne_real, zm, zr, p):
    n = h.shape[0]
    e_pad = src_p.shape[0] * CH
    we1 = p['We1']
    tsrc, tdst = _pre_call(h, we1[:H], we1[H:2 * H], p['be1'].reshape(1, H))
    gs, gd, qs, qd = _gather_kernel(n, e_pad)(tsrc, tdst, pos16, src_p,
                                              dst_p)
    mrm, mrr = _edge_call(ne_real, gs, gd, qs, qd, ea,
                          we1[2 * H + 1:], we1[2 * H].reshape(1, H),
                          p['We2'], p['be2'].reshape(1, H),
                          p['Wa'].reshape(1, H), p['ba'].reshape(1, 1),
                          p['Wc1'], p['bc1'].reshape(1, H),
                          p['Wc2'].reshape(1, H))
    pm, pr = _scatter_kernel(n, e_pad)(mrm, mrr, dst_p, zm, zr)
    wn1 = p['Wn1']
    return _node_call(h, pos16, pm[0], pm[1], pr[0], pr[1], wn1[:H],
                      wn1[H:], p['bn1'].reshape(1, H), p['Wn2'],
                      p['bn2'].reshape(1, H))


def _pad_edges(edge_index):
    e = edge_index.shape[1]
    e_pad = -(-e // (NWORK * CH)) * (NWORK * CH)
    pad = e_pad - e
    src = jnp.pad(edge_index[0].astype(jnp.int32), (0, pad)).reshape(-1, CH)
    dst = jnp.pad(edge_index[1].astype(jnp.int32), (0, pad)).reshape(-1, CH)
    return src, dst, e


def kernel(lig_x, lig_pos, lig_edge_index, lig_edge_attr, pocket_x,
           pocket_pos, pocket_edge_index, pocket_edge_attr, t, lig_batch,
           pocket_batch, L_We1, L_be1, L_We2, L_be2, L_Wa, L_ba, L_Wn1,
           L_bn1, L_Wn2, L_bn2, L_Wc1, L_bc1, L_Wc2, P_We1, P_be1, P_We2,
           P_be2, P_Wa, P_ba, P_Wn1, P_bn1, P_Wn2, P_bn2, P_Wc1, P_bc1,
           P_Wc2, W_emb_l, b_emb_l, W_emb_p, b_emb_p, Wt1, bt1, Wt2, bt2,
           Wp, bp, Wo, bo):
    pnames = ['We1', 'be1', 'We2', 'be2', 'Wa', 'ba', 'Wn1', 'bn1',
              'Wn2', 'bn2', 'Wc1', 'bc1', 'Wc2']
    lw = dict(zip(pnames, [L_We1, L_be1, L_We2, L_be2, L_Wa, L_ba, L_Wn1,
                           L_bn1, L_Wn2, L_bn2, L_Wc1, L_bc1, L_Wc2]))
    pw = dict(zip(pnames, [P_We1, P_be1, P_We2, P_be2, P_Wa, P_ba, P_Wn1,
                           P_bn1, P_Wn2, P_bn2, P_Wc1, P_bc1, P_Wc2]))

    n_p = pocket_x.shape[0]
    n_l = lig_x.shape[0]
    zm = jnp.zeros((n_p // NSUB, H), F32)
    zr = jnp.zeros((n_p // NSUB, PW), F32)

    hp = _emb_call(pocket_x, W_emb_p, b_emb_p.reshape(1, H))
    pp16 = jnp.pad(pocket_pos, ((0, 0), (0, PW - 3)))
    src_p, dst_p, ne_p = _pad_edges(pocket_edge_index)
    for i in range(P_We1.shape[0]):
        hp, pp16 = _egnn_layer(hp, pp16, src_p, dst_p, pocket_edge_attr,
                               ne_p, zm, zr,
                               {k: v[i] for k, v in pw.items()})

    psum = _pool_call(hp, pocket_batch.astype(jnp.int32).reshape(n_p, 1))
    tcond = _temb_cond_call(t.astype(jnp.int32).reshape(NBATCH, 1),
                            Wt1, bt1.reshape(1, H), Wt2, bt2.reshape(1, H),
                            psum, Wp, bp.reshape(1, H))

    h = _emb_lig_call(lig_x, lig_batch.astype(jnp.int32).reshape(n_l, 1),
                      W_emb_l, b_emb_l.reshape(1, H), tcond)
    pl16 = jnp.pad(lig_pos, ((0, 0), (0, PW - 3)))
    src_l, dst_l, ne_l = _pad_edges(lig_edge_index)
    pos16 = pl16
    for i in range(L_We1.shape[0]):
        h, pos16 = _egnn_layer(h, pos16, src_l, dst_l, lig_edge_attr,
                               ne_l, zm, zr,
                               {k: v[i] for k, v in lw.items()})

    type_pred, coord16 = _final_call(h, Wo, bo.reshape(1, NT), pos16, pl16)
    return type_pred, coord16[:, :3]


# speedup vs baseline: 2.4899x; 1.1684x over previous
import functools

import jax
import jax.numpy as jnp
from jax import lax
from jax.experimental import pallas as pl
from jax.experimental.pallas import tpu as pltpu
from jax.experimental.pallas import tpu_sc as plsc

F32 = jnp.float32
H = 128
ED = 16
NT = 10
TDIM = 128
NBATCH = 64
WID = 144
PW = 16
NB = 1000
EB = 1024
NCORES = 2
NSUB = 16
NWORK = NCORES * NSUB
CH = 128

_HI = lax.Precision.DEFAULT
BF16 = jnp.bfloat16


def _dot(a, b):
    return jnp.dot(a, b, precision=_HI)


def _silu(x):
    return x * jax.nn.sigmoid(x)


def _mesh():
    return plsc.VectorSubcoreMesh(core_axis_name="c", subcore_axis_name="s",
                                  num_cores=NCORES, num_subcores=NSUB)


def _emb_body(x_ref, w_ref, b_ref, o_ref):
    o_ref[...] = _dot(x_ref[...], w_ref[...]) + b_ref[...]


def _emb_call(x, w, b):
    n = x.shape[0]
    return pl.pallas_call(
        _emb_body,
        grid=(n // NB,),
        in_specs=[pl.BlockSpec((NB, x.shape[1]), lambda i: (i, 0)),
                  pl.BlockSpec(w.shape, lambda i: (0, 0)),
                  pl.BlockSpec(b.shape, lambda i: (0, 0))],
        out_specs=pl.BlockSpec((NB, w.shape[1]), lambda i: (i, 0)),
        out_shape=jax.ShapeDtypeStruct((n, w.shape[1]), F32),
    )(x, w, b)


def _emb_lig_body(x_ref, bt_ref, w_ref, b_ref, tc_ref, o_ref):
    oh = (bt_ref[...] == lax.broadcasted_iota(jnp.int32, (1, NBATCH), 1)
          ).astype(F32)
    o_ref[...] = (_dot(x_ref[...], w_ref[...]) + b_ref[...]
                  + _dot(oh, tc_ref[...]))


def _emb_lig_call(x, batch2d, w, b, tcond):
    n = x.shape[0]
    return pl.pallas_call(
        _emb_lig_body,
        grid=(n // NB,),
        in_specs=[pl.BlockSpec((NB, x.shape[1]), lambda i: (i, 0)),
                  pl.BlockSpec((NB, 1), lambda i: (i, 0)),
                  pl.BlockSpec(w.shape, lambda i: (0, 0)),
                  pl.BlockSpec(b.shape, lambda i: (0, 0)),
                  pl.BlockSpec(tcond.shape, lambda i: (0, 0))],
        out_specs=pl.BlockSpec((NB, H), lambda i: (i, 0)),
        out_shape=jax.ShapeDtypeStruct((n, H), F32),
    )(x, batch2d, w, b, tcond)


def _pool_body(h_ref, bt_ref, o_ref):
    i = pl.program_id(0)
    oh = (bt_ref[...] == lax.broadcasted_iota(jnp.int32, (1, NBATCH), 1)
          ).astype(F32)
    ssum = lax.dot_general(oh, h_ref[...], (((0,), (0,)), ((), ())),
                           precision=_HI)
    lane = lax.broadcasted_iota(jnp.int32, (NB, PW), 1)
    ones0 = (lane == 0).astype(F32)
    scnt = lax.dot_general(oh, ones0, (((0,), (0,)), ((), ())),
                           precision=_HI)

    @pl.when(i == 0)
    def _():
        o_ref[:, :H] = ssum
        o_ref[:, H:] = scnt

    @pl.when(i > 0)
    def _():
        o_ref[:, :H] += ssum
        o_ref[:, H:] += scnt


def _pool_call(h, batch2d):
    n = h.shape[0]
    return pl.pallas_call(
        _pool_body,
        grid=(n // NB,),
        in_specs=[pl.BlockSpec((NB, H), lambda i: (i, 0)),
                  pl.BlockSpec((NB, 1), lambda i: (i, 0))],
        out_specs=pl.BlockSpec((NBATCH, H + PW), lambda i: (0, 0)),
        out_shape=jax.ShapeDtypeStruct((NBATCH, H + PW), F32),
    )(h, batch2d)


def _temb_cond_body(t_ref, wt1, bt1, wt2, bt2, ps_ref, wp, bp, o_ref):
    t = t_ref[...].astype(F32)
    half = TDIM // 2
    k = lax.broadcasted_iota(jnp.int32, (1, half), 1).astype(F32)
    freqs = jnp.exp(-jnp.log(10000.0) * k / float(half))
    args = t * freqs
    temb = jnp.concatenate([jnp.sin(args), jnp.cos(args)], axis=1)
    temb = _silu(_dot(temb, wt1[...]) + bt1[...])
    temb = _dot(temb, wt2[...]) + bt2[...]
    ps = ps_ref[...]
    lane = lax.broadcasted_iota(jnp.int32, (NBATCH, PW), 1)
    cnt = jnp.sum(ps[:, H:] * (lane == 0).astype(F32), axis=1, keepdims=True)
    pooled = ps[:, :H] / (cnt + 1e-6)
    cond = _silu(_dot(pooled, wp[...]) + bp[...])
    o_ref[...] = temb + cond


def _temb_cond_call(t2d, wt1, bt1, wt2, bt2, psum, wp, bp):
    return pl.pallas_call(
        _temb_cond_body,
        out_shape=jax.ShapeDtypeStruct((NBATCH, H), F32),
    )(t2d, wt1, bt1, wt2, bt2, psum, wp, bp)


def _pre_body(h_ref, ws, wd, be1, ts_ref, td_ref):
    hb = h_ref[...]
    ts_ref[...] = _dot(hb, ws[...]) + be1[...]
    td_ref[...] = _dot(hb, wd[...])


def _pre_call(h, ws, wd, be1):
    n = h.shape[0]
    return pl.pallas_call(
        _pre_body,
        grid=(n // NB,),
        in_specs=[pl.BlockSpec((NB, H), lambda i: (i, 0)),
                  pl.BlockSpec((H, H), lambda i: (0, 0)),
                  pl.BlockSpec((H, H), lambda i: (0, 0)),
                  pl.BlockSpec((1, H), lambda i: (0, 0))],
        out_specs=[pl.BlockSpec((NB, H), lambda i: (i, 0)),
                   pl.BlockSpec((NB, H), lambda i: (i, 0))],
        out_shape=[jax.ShapeDtypeStruct((n, H), F32),
                   jax.ShapeDtypeStruct((n, H), F32)],
    )(h, ws, wd, be1)


def _edge_body(ne_real, row0, gs_ref, gd_ref, ps_ref, pd_ref, ea_ref, wea,
               wd2, we2, be2, wa, ba, wc1, bc1, wc2, om_ref, or_ref):
    i = pl.program_id(0)
    gs = gs_ref[...]
    gd = gd_ref[...]
    rel = ps_ref[...] - pd_ref[...]
    d2 = jnp.sum(rel * rel, axis=1, keepdims=True)
    m1 = _silu(gs + gd + d2 * wd2[...]
               + _dot(ea_ref[...], wea[...]))
    m2 = _silu(_dot(m1, we2[...]) + be2[...])
    gate = jax.nn.sigmoid(
        jnp.sum(m2 * wa[...], axis=1, keepdims=True) + ba[...])
    m = m2 * gate
    c2 = _silu(_dot(m, wc1[...]) + bc1[...])
    c = jnp.sum(c2 * wc2[...], axis=1, keepdims=True)
    row = row0 + i * EB + lax.broadcasted_iota(jnp.int32, (EB, 1), 0)
    valid = (row < ne_real).astype(F32)
    lane = lax.broadcasted_iota(jnp.int32, (EB, PW), 1)
    deg1 = (lane == 3).astype(F32)
    om_ref[...] = m * valid
    or_ref[...] = (rel * c + deg1) * valid


def _edge_call(ne_real, row0, gs, gd, ps, pd, ea, wea, wd2, we2, be2, wa,
               ba, wc1, bc1, wc2):
    e_pad = gs.shape[0]
    nblk = e_pad // EB
    last = ne_real // EB - (1 if ne_real % EB == 0 else 0)
    blk0 = row0 // EB
    body = functools.partial(_edge_body, ne_real, row0)

    def ea_map(i):
        return (jnp.minimum(blk0 + i, last), 0)

    return pl.pallas_call(
        body,
        grid=(nblk,),
        in_specs=[pl.BlockSpec((EB, H), lambda i: (i, 0)),
                  pl.BlockSpec((EB, H), lambda i: (i, 0)),
                  pl.BlockSpec((EB, PW), lambda i: (i, 0)),
                  pl.BlockSpec((EB, PW), lambda i: (i, 0)),
                  pl.BlockSpec((EB, ED), ea_map),
                  pl.BlockSpec((ED, H), lambda i: (0, 0)),
                  pl.BlockSpec((1, H), lambda i: (0, 0)),
                  pl.BlockSpec((H, H), lambda i: (0, 0)),
                  pl.BlockSpec((1, H), lambda i: (0, 0)),
                  pl.BlockSpec((1, H), lambda i: (0, 0)),
                  pl.BlockSpec((1, 1), lambda i: (0, 0)),
                  pl.BlockSpec((H, H), lambda i: (0, 0)),
                  pl.BlockSpec((1, H), lambda i: (0, 0)),
                  pl.BlockSpec((1, H), lambda i: (0, 0))],
        out_specs=[pl.BlockSpec((EB, H), lambda i: (i, 0)),
                   pl.BlockSpec((EB, PW), lambda i: (i, 0))],
        out_shape=[jax.ShapeDtypeStruct((e_pad, H), F32),
                   jax.ShapeDtypeStruct((e_pad, PW), F32)],
    )(gs, gd, ps, pd, ea, wea, wd2, we2, be2, wa, ba, wc1, bc1, wc2)


def _node_body(h_ref, p_ref, a0m_ref, a1m_ref, a2m_ref, a3m_ref, a0r_ref,
               a1r_ref, a2r_ref, a3r_ref, wn1h, wn1a, bn1, wn2, bn2,
               ho_ref, po_ref):
    h = h_ref[...]
    aggm = (a0m_ref[...] + a1m_ref[...]) + (a2m_ref[...] + a3m_ref[...])
    agg16 = (a0r_ref[...] + a1r_ref[...]) + (a2r_ref[...] + a3r_ref[...])
    lane = lax.broadcasted_iota(jnp.int32, (NB, PW), 1)
    deg = jnp.sum(agg16 * (lane == 3).astype(F32), axis=1, keepdims=True)
    relc = jnp.where(lane < 3, agg16, 0.0)
    u = _silu(_dot(h, wn1h[...]) + _dot(aggm, wn1a[...]) + bn1[...])
    ho_ref[...] = h + _dot(u, wn2[...]) + bn2[...]
    po_ref[...] = p_ref[...] + relc / (deg + 1.0)


def _node_call(h, pos16, ams, ars, wn1h, wn1a, bn1, wn2, bn2):
    n = h.shape[0]
    return pl.pallas_call(
        _node_body,
        grid=(n // NB,),
        in_specs=[pl.BlockSpec((NB, H), lambda i: (i, 0)),
                  pl.BlockSpec((NB, PW), lambda i: (i, 0)),
                  pl.BlockSpec((NB, H), lambda i: (i, 0)),
                  pl.BlockSpec((NB, H), lambda i: (i, 0)),
                  pl.BlockSpec((NB, H), lambda i: (i, 0)),
                  pl.BlockSpec((NB, H), lambda i: (i, 0)),
                  pl.BlockSpec((NB, PW), lambda i: (i, 0)),
                  pl.BlockSpec((NB, PW), lambda i: (i, 0)),
                  pl.BlockSpec((NB, PW), lambda i: (i, 0)),
                  pl.BlockSpec((NB, PW), lambda i: (i, 0)),
                  pl.BlockSpec((H, H), lambda i: (0, 0)),
                  pl.BlockSpec((H, H), lambda i: (0, 0)),
                  pl.BlockSpec((1, H), lambda i: (0, 0)),
                  pl.BlockSpec((H, H), lambda i: (0, 0)),
                  pl.BlockSpec((1, H), lambda i: (0, 0))],
        out_specs=[pl.BlockSpec((NB, H), lambda i: (i, 0)),
                   pl.BlockSpec((NB, PW), lambda i: (i, 0))],
        out_shape=[jax.ShapeDtypeStruct((n, H), F32),
                   jax.ShapeDtypeStruct((n, PW), F32)],
    )(h, pos16, *ams, *ars, wn1h, wn1a, bn1, wn2, bn2)


def _final_body(h_ref, wo, bo, p_ref, p0_ref, t_ref, c_ref):
    t_ref[...] = _dot(h_ref[...], wo[...]) + bo[...]
    c_ref[...] = p_ref[...] - p0_ref[...]


def _final_call(h, wo, bo, pos16, pos016):
    n = h.shape[0]
    return pl.pallas_call(
        _final_body,
        grid=(n // NB,),
        in_specs=[pl.BlockSpec((NB, H), lambda i: (i, 0)),
                  pl.BlockSpec((H, NT), lambda i: (0, 0)),
                  pl.BlockSpec((1, NT), lambda i: (0, 0)),
                  pl.BlockSpec((NB, PW), lambda i: (i, 0)),
                  pl.BlockSpec((NB, PW), lambda i: (i, 0))],
        out_specs=[pl.BlockSpec((NB, NT), lambda i: (i, 0)),
                   pl.BlockSpec((NB, PW), lambda i: (i, 0))],
        out_shape=[jax.ShapeDtypeStruct((n, NT), F32),
                   jax.ShapeDtypeStruct((n, PW), F32)],
    )(h, wo, bo, pos16, pos016)


@functools.lru_cache(maxsize=None)
def _gather_kernel(n_rows, e_pad):
    chunks = e_pad // (NWORK * CH)
    per_w = chunks * CH

    @functools.partial(
        pl.kernel,
        out_type=(jax.ShapeDtypeStruct((e_pad, H), F32),
                  jax.ShapeDtypeStruct((e_pad, H), F32),
                  jax.ShapeDtypeStruct((e_pad, PW), F32),
                  jax.ShapeDtypeStruct((e_pad, PW), F32)),
        mesh=_mesh(),
        scratch_types=[pltpu.VMEM((chunks, CH), jnp.int32),
                       pltpu.VMEM((chunks, CH), jnp.int32),
                       pltpu.VMEM((2, CH, H), F32),
                       pltpu.VMEM((2, CH, H), F32),
                       pltpu.VMEM((2, CH, PW), F32),
                       pltpu.VMEM((2, CH, PW), F32),
                       pltpu.SemaphoreType.DMA,
                       pltpu.SemaphoreType.DMA,
                       pltpu.SemaphoreType.DMA,
                       pltpu.SemaphoreType.DMA],
        compiler_params=pltpu.CompilerParams(use_tc_tiling_on_sc=False))
    def k(tsrc, tdst, pos, srci, dsti, gs, gd, qs, qd, idx_s_ref, idx_d_ref,
          row_s, row_d, pr_s, pr_d, sem_g0, sem_g1, sem_w0, sem_w1):
        w = lax.axis_index("c") * NSUB + lax.axis_index("s")
        pltpu.sync_copy(srci.at[pl.ds(w * chunks, chunks)], idx_s_ref)
        pltpu.sync_copy(dsti.at[pl.ds(w * chunks, chunks)], idx_d_ref)
        sem_g = [sem_g0, sem_g1]
        sem_w = [sem_w0, sem_w1]

        def body(i, carry):
            base = w * per_w + i * CH

            def run(p):
                @pl.when(i >= 2)
                def _():
                    pltpu.make_async_copy(
                        row_s.at[p], gs.at[pl.ds(base, CH)],
                        sem_w[p]).wait()
                    pltpu.make_async_copy(
                        row_d.at[p], gd.at[pl.ds(base, CH)],
                        sem_w[p]).wait()
                    pltpu.make_async_copy(
                        pr_s.at[p], qs.at[pl.ds(base, CH)],
                        sem_w[p]).wait()
                    pltpu.make_async_copy(
                        pr_d.at[p], qd.at[pl.ds(base, CH)],
                        sem_w[p]).wait()
                cps = [
                    pltpu.async_copy(tsrc.at[idx_s_ref.at[i]], row_s.at[p],
                                     sem_g[p]),
                    pltpu.async_copy(tdst.at[idx_d_ref.at[i]], row_d.at[p],
                                     sem_g[p]),
                    pltpu.async_copy(pos.at[idx_s_ref.at[i]], pr_s.at[p],
                                     sem_g[p]),
                    pltpu.async_copy(pos.at[idx_d_ref.at[i]], pr_d.at[p],
                                     sem_g[p]),
                ]
                for c in cps:
                    c.wait()
                pltpu.async_copy(row_s.at[p], gs.at[pl.ds(base, CH)],
                                 sem_w[p])
                pltpu.async_copy(row_d.at[p], gd.at[pl.ds(base, CH)],
                                 sem_w[p])
                pltpu.async_copy(pr_s.at[p], qs.at[pl.ds(base, CH)],
                                 sem_w[p])
                pltpu.async_copy(pr_d.at[p], qd.at[pl.ds(base, CH)],
                                 sem_w[p])

            @pl.when(i % 2 == 0)
            def _():
                run(0)

            @pl.when(i % 2 == 1)
            def _():
                run(1)

            return carry

        lax.fori_loop(0, chunks, body, 0)
        for p in range(2):
            pltpu.make_async_copy(row_s.at[p], gs.at[pl.ds(0, CH)],
                                  sem_w[p]).wait()
            pltpu.make_async_copy(row_d.at[p], gd.at[pl.ds(0, CH)],
                                  sem_w[p]).wait()
            pltpu.make_async_copy(pr_s.at[p], qs.at[pl.ds(0, CH)],
                                  sem_w[p]).wait()
            pltpu.make_async_copy(pr_d.at[p], qd.at[pl.ds(0, CH)],
                                  sem_w[p]).wait()

    return k


@functools.lru_cache(maxsize=None)
def _scatter_kernel(n_rows, e_pad):
    chunks = e_pad // (NWORK * CH)
    per_w = chunks * CH
    rpt = n_rows // NSUB
    oc = 5
    ocs = rpt // oc

    @functools.partial(
        pl.kernel,
        out_type=(jax.ShapeDtypeStruct((NCORES, n_rows, H), F32),
                  jax.ShapeDtypeStruct((NCORES, n_rows, PW), F32)),
        mesh=_mesh(),
        scratch_types=[pltpu.VMEM_SHARED((n_rows, H), F32),
                       pltpu.VMEM_SHARED((n_rows, PW), F32),
                       pltpu.VMEM((2, CH), jnp.int32),
                       pltpu.VMEM((2, CH, H), F32),
                       pltpu.VMEM((2, CH, PW), F32),
                       pltpu.SemaphoreType.DMA,
                       pltpu.SemaphoreType.DMA,
                       pltpu.SemaphoreType.DMA,
                       pltpu.SemaphoreType.DMA],
        compiler_params=pltpu.CompilerParams(use_tc_tiling_on_sc=False))
    def k(mrm, mrr, dsti, zm, zr, outm, outr, accm, accr, idx, valm, valr,
          sem_a0, sem_a1, sem_l0, sem_l1):
        cid = lax.axis_index("c")
        sid = lax.axis_index("s")
        w = cid * NSUB + sid
        pltpu.sync_copy(zm, accm.at[pl.ds(sid * rpt, rpt)])
        pltpu.sync_copy(zr, accr.at[pl.ds(sid * rpt, rpt)])
        plsc.subcore_barrier()
        sem_a = [sem_a0, sem_a1]
        sem_l = [sem_l0, sem_l1]

        def body(i, carry):
            base = w * per_w + i * CH

            def run(p):
                @pl.when(i >= 2)
                def _():
                    pltpu.make_async_copy(valm.at[p], accm.at[idx.at[p]],
                                          sem_a[p]).wait()
                    pltpu.make_async_copy(valr.at[p], accr.at[idx.at[p]],
                                          sem_a[p]).wait()
                cps = [
                    pltpu.async_copy(dsti.at[w * chunks + i], idx.at[p],
                                     sem_l[p]),
                    pltpu.async_copy(mrm.at[pl.ds(base, CH)], valm.at[p],
                                     sem_l[p]),
                    pltpu.async_copy(mrr.at[pl.ds(base, CH)], valr.at[p],
                                     sem_l[p]),
                ]
                for c in cps:
                    c.wait()
                pltpu.async_copy(valm.at[p], accm.at[idx.at[p]], sem_a[p],
                                 add=True)
                pltpu.async_copy(valr.at[p], accr.at[idx.at[p]], sem_a[p],
                                 add=True)

            @pl.when(i % 2 == 0)
            def _():
                run(0)

            @pl.when(i % 2 == 1)
            def _():
                run(1)

            return carry

        lax.fori_loop(0, chunks, body, 0)
        for p in range(2):
            pltpu.make_async_copy(valm.at[p], accm.at[idx.at[p]],
                                  sem_a[p]).wait()
            pltpu.make_async_copy(valr.at[p], accr.at[idx.at[p]],
                                  sem_a[p]).wait()
        plsc.subcore_barrier()

        def flush(j, carry):
            s = sid * rpt + j * ocs
            pltpu.sync_copy(accm.at[pl.ds(s, ocs)],
                            outm.at[cid, pl.ds(s, ocs)])
            pltpu.sync_copy(accr.at[pl.ds(s, ocs)],
                            outr.at[cid, pl.ds(s, ocs)])
            return carry

        lax.fori_loop(0, oc, flush, 0)

    return k


def _egnn_layer(h, pos16, src_p, dst_p, ea, ne_real, zm, zr, p):
    n = h.shape[0]
    nrows = src_p.shape[0]
    e_half = (nrows // 2) * CH
    we1 = p['We1']
    tsrc, tdst = _pre_call(h, we1[:H], we1[H:2 * H], p['be1'].reshape(1, H))
    hrows = nrows // 2
    gath = _gather_kernel(n, e_half)
    scat = _scatter_kernel(n, e_half)
    halves = []
    for hf in range(2):
        src_h = src_p[hf * hrows:(hf + 1) * hrows]
        dst_h = dst_p[hf * hrows:(hf + 1) * hrows]
        halves.append((src_h, dst_h))
    gres = [gath(tsrc, tdst, pos16, s, d) for s, d in halves]
    mres = []
    for hf, (gs, gd, qs, qd) in enumerate(gres):
        mres.append(_edge_call(ne_real, hf * e_half, gs, gd, qs, qd, ea,
                               we1[2 * H + 1:], we1[2 * H].reshape(1, H),
                               p['We2'], p['be2'].reshape(1, H),
                               p['Wa'].reshape(1, H), p['ba'].reshape(1, 1),
                               p['Wc1'], p['bc1'].reshape(1, H),
                               p['Wc2'].reshape(1, H)))
    ams, ars = [], []
    for hf, (mrm, mrr) in enumerate(mres):
        pm, pr = scat(mrm, mrr, halves[hf][1], zm, zr)
        ams.extend([pm[0], pm[1]])
        ars.extend([pr[0], pr[1]])
    wn1 = p['Wn1']
    return _node_call(h, pos16, ams, ars, wn1[:H], wn1[H:],
                      p['bn1'].reshape(1, H), p['Wn2'],
                      p['bn2'].reshape(1, H))


def _pad_edges(edge_index):
    e = edge_index.shape[1]
    e_pad = -(-e // (NWORK * CH)) * (NWORK * CH)
    pad = e_pad - e
    src = jnp.pad(edge_index[0].astype(jnp.int32), (0, pad)).reshape(-1, CH)
    dst = jnp.pad(edge_index[1].astype(jnp.int32), (0, pad)).reshape(-1, CH)
    return src, dst, e


def kernel(lig_x, lig_pos, lig_edge_index, lig_edge_attr, pocket_x,
           pocket_pos, pocket_edge_index, pocket_edge_attr, t, lig_batch,
           pocket_batch, L_We1, L_be1, L_We2, L_be2, L_Wa, L_ba, L_Wn1,
           L_bn1, L_Wn2, L_bn2, L_Wc1, L_bc1, L_Wc2, P_We1, P_be1, P_We2,
           P_be2, P_Wa, P_ba, P_Wn1, P_bn1, P_Wn2, P_bn2, P_Wc1, P_bc1,
           P_Wc2, W_emb_l, b_emb_l, W_emb_p, b_emb_p, Wt1, bt1, Wt2, bt2,
           Wp, bp, Wo, bo):
    pnames = ['We1', 'be1', 'We2', 'be2', 'Wa', 'ba', 'Wn1', 'bn1',
              'Wn2', 'bn2', 'Wc1', 'bc1', 'Wc2']
    lw = dict(zip(pnames, [L_We1, L_be1, L_We2, L_be2, L_Wa, L_ba, L_Wn1,
                           L_bn1, L_Wn2, L_bn2, L_Wc1, L_bc1, L_Wc2]))
    pw = dict(zip(pnames, [P_We1, P_be1, P_We2, P_be2, P_Wa, P_ba, P_Wn1,
                           P_bn1, P_Wn2, P_bn2, P_Wc1, P_bc1, P_Wc2]))

    n_p = pocket_x.shape[0]
    n_l = lig_x.shape[0]
    zm = jnp.zeros((n_p // NSUB, H), F32)
    zr = jnp.zeros((n_p // NSUB, PW), F32)

    hp = _emb_call(pocket_x, W_emb_p, b_emb_p.reshape(1, H))
    pp16 = jnp.pad(pocket_pos, ((0, 0), (0, PW - 3)))
    src_p, dst_p, ne_p = _pad_edges(pocket_edge_index)
    for i in range(P_We1.shape[0]):
        hp, pp16 = _egnn_layer(hp, pp16, src_p, dst_p, pocket_edge_attr,
                               ne_p, zm, zr,
                               {k: v[i] for k, v in pw.items()})

    psum = _pool_call(hp, pocket_batch.astype(jnp.int32).reshape(n_p, 1))
    tcond = _temb_cond_call(t.astype(jnp.int32).reshape(NBATCH, 1),
                            Wt1, bt1.reshape(1, H), Wt2, bt2.reshape(1, H),
                            psum, Wp, bp.reshape(1, H))

    h = _emb_lig_call(lig_x, lig_batch.astype(jnp.int32).reshape(n_l, 1),
                      W_emb_l, b_emb_l.reshape(1, H), tcond)
    pl16 = jnp.pad(lig_pos, ((0, 0), (0, PW - 3)))
    src_l, dst_l, ne_l = _pad_edges(lig_edge_index)
    pos16 = pl16
    for i in range(L_We1.shape[0]):
        h, pos16 = _egnn_layer(h, pos16, src_l, dst_l, lig_edge_attr,
                               ne_l, zm, zr,
                               {k: v[i] for k, v in lw.items()})

    type_pred, coord16 = _final_call(h, Wo, bo.reshape(1, NT), pos16, pl16)
    return type_pred, coord16[:, :3]


# speedup vs baseline: 2.8767x; 1.1554x over previous
import functools

import jax
import jax.numpy as jnp
from jax import lax
from jax.experimental import pallas as pl
from jax.experimental.pallas import tpu as pltpu
from jax.experimental.pallas import tpu_sc as plsc

F32 = jnp.float32
H = 128
ED = 16
NT = 10
TDIM = 128
NBATCH = 64
WID = 144
PW = 16
NB = 1000
EB = 1024
NCORES = 2
NSUB = 16
NWORK = NCORES * NSUB
CH = 128

_HI = lax.Precision.DEFAULT
BF16 = jnp.bfloat16


def _dot(a, b):
    return jnp.dot(a, b, precision=_HI)


def _silu(x):
    return x * jax.nn.sigmoid(x)


def _mesh():
    return plsc.VectorSubcoreMesh(core_axis_name="c", subcore_axis_name="s",
                                  num_cores=NCORES, num_subcores=NSUB)


def _emb_body(x_ref, w_ref, b_ref, o_ref):
    o_ref[...] = _dot(x_ref[...], w_ref[...]) + b_ref[...]


def _emb_call(x, w, b):
    n = x.shape[0]
    return pl.pallas_call(
        _emb_body,
        grid=(n // NB,),
        in_specs=[pl.BlockSpec((NB, x.shape[1]), lambda i: (i, 0)),
                  pl.BlockSpec(w.shape, lambda i: (0, 0)),
                  pl.BlockSpec(b.shape, lambda i: (0, 0))],
        out_specs=pl.BlockSpec((NB, w.shape[1]), lambda i: (i, 0)),
        out_shape=jax.ShapeDtypeStruct((n, w.shape[1]), F32),
    )(x, w, b)


def _emb_lig_body(x_ref, bt_ref, w_ref, b_ref, tc_ref, o_ref):
    oh = (bt_ref[...] == lax.broadcasted_iota(jnp.int32, (1, NBATCH), 1)
          ).astype(F32)
    o_ref[...] = (_dot(x_ref[...], w_ref[...]) + b_ref[...]
                  + _dot(oh, tc_ref[...]))


def _emb_lig_call(x, batch2d, w, b, tcond):
    n = x.shape[0]
    return pl.pallas_call(
        _emb_lig_body,
        grid=(n // NB,),
        in_specs=[pl.BlockSpec((NB, x.shape[1]), lambda i: (i, 0)),
                  pl.BlockSpec((NB, 1), lambda i: (i, 0)),
                  pl.BlockSpec(w.shape, lambda i: (0, 0)),
                  pl.BlockSpec(b.shape, lambda i: (0, 0)),
                  pl.BlockSpec(tcond.shape, lambda i: (0, 0))],
        out_specs=pl.BlockSpec((NB, H), lambda i: (i, 0)),
        out_shape=jax.ShapeDtypeStruct((n, H), F32),
    )(x, batch2d, w, b, tcond)


def _pool_body(h_ref, bt_ref, o_ref):
    i = pl.program_id(0)
    oh = (bt_ref[...] == lax.broadcasted_iota(jnp.int32, (1, NBATCH), 1)
          ).astype(F32)
    ssum = lax.dot_general(oh, h_ref[...], (((0,), (0,)), ((), ())),
                           precision=_HI)
    lane = lax.broadcasted_iota(jnp.int32, (NB, PW), 1)
    ones0 = (lane == 0).astype(F32)
    scnt = lax.dot_general(oh, ones0, (((0,), (0,)), ((), ())),
                           precision=_HI)

    @pl.when(i == 0)
    def _():
        o_ref[:, :H] = ssum
        o_ref[:, H:] = scnt

    @pl.when(i > 0)
    def _():
        o_ref[:, :H] += ssum
        o_ref[:, H:] += scnt


def _pool_call(h, batch2d):
    n = h.shape[0]
    return pl.pallas_call(
        _pool_body,
        grid=(n // NB,),
        in_specs=[pl.BlockSpec((NB, H), lambda i: (i, 0)),
                  pl.BlockSpec((NB, 1), lambda i: (i, 0))],
        out_specs=pl.BlockSpec((NBATCH, H + PW), lambda i: (0, 0)),
        out_shape=jax.ShapeDtypeStruct((NBATCH, H + PW), F32),
    )(h, batch2d)


def _temb_cond_body(t_ref, wt1, bt1, wt2, bt2, ps_ref, wp, bp, o_ref):
    t = t_ref[...].astype(F32)
    half = TDIM // 2
    k = lax.broadcasted_iota(jnp.int32, (1, half), 1).astype(F32)
    freqs = jnp.exp(-jnp.log(10000.0) * k / float(half))
    args = t * freqs
    temb = jnp.concatenate([jnp.sin(args), jnp.cos(args)], axis=1)
    temb = _silu(_dot(temb, wt1[...]) + bt1[...])
    temb = _dot(temb, wt2[...]) + bt2[...]
    ps = ps_ref[...]
    lane = lax.broadcasted_iota(jnp.int32, (NBATCH, PW), 1)
    cnt = jnp.sum(ps[:, H:] * (lane == 0).astype(F32), axis=1, keepdims=True)
    pooled = ps[:, :H] / (cnt + 1e-6)
    cond = _silu(_dot(pooled, wp[...]) + bp[...])
    o_ref[...] = temb + cond


def _temb_cond_call(t2d, wt1, bt1, wt2, bt2, psum, wp, bp):
    return pl.pallas_call(
        _temb_cond_body,
        out_shape=jax.ShapeDtypeStruct((NBATCH, H), F32),
    )(t2d, wt1, bt1, wt2, bt2, psum, wp, bp)


def _pre_body(h_ref, ws, wd, be1, ts_ref, td_ref):
    hb = h_ref[...]
    ts_ref[...] = _dot(hb, ws[...]) + be1[...]
    td_ref[...] = _dot(hb, wd[...])


def _pre_call(h, ws, wd, be1):
    n = h.shape[0]
    return pl.pallas_call(
        _pre_body,
        grid=(n // NB,),
        in_specs=[pl.BlockSpec((NB, H), lambda i: (i, 0)),
                  pl.BlockSpec((H, H), lambda i: (0, 0)),
                  pl.BlockSpec((H, H), lambda i: (0, 0)),
                  pl.BlockSpec((1, H), lambda i: (0, 0))],
        out_specs=[pl.BlockSpec((NB, H), lambda i: (i, 0)),
                   pl.BlockSpec((NB, H), lambda i: (i, 0))],
        out_shape=[jax.ShapeDtypeStruct((n, H), F32),
                   jax.ShapeDtypeStruct((n, H), F32)],
    )(h, ws, wd, be1)


def _edge_body(ne_real, row0, gs_ref, gd_ref, ps_ref, pd_ref, ea_ref, wea,
               wd2, we2, be2, wa, ba, wc1, bc1, wc2, om_ref, or_ref):
    i = pl.program_id(0)
    gs = gs_ref[...]
    gd = gd_ref[...]
    eb8 = EB // 8
    relp = ps_ref[...] - pd_ref[...]
    sqp = relp * relp
    lane_g = lax.broadcasted_iota(jnp.int32, (128, 8), 0) // PW
    sel_g = (lane_g == lax.broadcasted_iota(jnp.int32, (128, 8), 1)
             ).astype(F32)
    d2blk = _dot(sqp, sel_g)
    e_i0 = lax.broadcasted_iota(jnp.int32, (EB, 128), 0)
    s1 = (e_i0 // 8 == lax.broadcasted_iota(jnp.int32, (EB, 128), 1)
          ).astype(F32)
    m2sel = (lax.broadcasted_iota(jnp.int32, (EB, 8), 0) % 8 ==
             lax.broadcasted_iota(jnp.int32, (EB, 8), 1)).astype(F32)
    d2 = jnp.sum(_dot(s1[:, :eb8], d2blk) * m2sel, axis=1, keepdims=True)
    m1 = _silu(gs + gd + d2 * wd2[...]
               + _dot(ea_ref[...], wea[...]))
    m2 = _silu(_dot(m1, we2[...]) + be2[...])
    gate = jax.nn.sigmoid(
        jnp.sum(m2 * wa[...], axis=1, keepdims=True) + ba[...])
    m = m2 * gate
    c2 = _silu(_dot(m, wc1[...]) + bc1[...])
    c = jnp.sum(c2 * wc2[...], axis=1, keepdims=True)
    row = row0 + i * EB + lax.broadcasted_iota(jnp.int32, (EB, 1), 0)
    valid = (row < ne_real).astype(F32)
    cblk = lax.dot_general(s1[:, :eb8], c * m2sel, (((0,), (0,)), ((), ())),
                           precision=_HI)
    sel_e = (lax.broadcasted_iota(jnp.int32, (8, 128), 0) ==
             lax.broadcasted_iota(jnp.int32, (8, 128), 1) // PW).astype(F32)
    cpack = _dot(cblk, sel_e)
    lane128 = lax.broadcasted_iota(jnp.int32, (eb8, 128), 1)
    deg1p = (lane128 % PW == 3).astype(F32)
    ep = (row0 + i * EB + 8 * lax.broadcasted_iota(jnp.int32, (eb8, 128), 0)
          + lane128 // PW)
    validp = (ep < ne_real).astype(F32)
    om_ref[...] = m * valid
    or_ref[...] = (relp * cpack + deg1p) * validp


def _edge_call(ne_real, row0, gs, gd, ps, pd, ea, wea, wd2, we2, be2, wa,
               ba, wc1, bc1, wc2):
    e_pad = gs.shape[0]
    nblk = e_pad // EB
    last = ne_real // EB - (1 if ne_real % EB == 0 else 0)
    blk0 = row0 // EB
    body = functools.partial(_edge_body, ne_real, row0)

    def ea_map(i):
        return (jnp.minimum(blk0 + i, last), 0)

    return pl.pallas_call(
        body,
        grid=(nblk,),
        in_specs=[pl.BlockSpec((EB, H), lambda i: (i, 0)),
                  pl.BlockSpec((EB, H), lambda i: (i, 0)),
                  pl.BlockSpec((EB // 8, 128), lambda i: (i, 0)),
                  pl.BlockSpec((EB // 8, 128), lambda i: (i, 0)),
                  pl.BlockSpec((EB, ED), ea_map),
                  pl.BlockSpec((ED, H), lambda i: (0, 0)),
                  pl.BlockSpec((1, H), lambda i: (0, 0)),
                  pl.BlockSpec((H, H), lambda i: (0, 0)),
                  pl.BlockSpec((1, H), lambda i: (0, 0)),
                  pl.BlockSpec((1, H), lambda i: (0, 0)),
                  pl.BlockSpec((1, 1), lambda i: (0, 0)),
                  pl.BlockSpec((H, H), lambda i: (0, 0)),
                  pl.BlockSpec((1, H), lambda i: (0, 0)),
                  pl.BlockSpec((1, H), lambda i: (0, 0))],
        out_specs=[pl.BlockSpec((EB, H), lambda i: (i, 0)),
                   pl.BlockSpec((EB // 8, 128), lambda i: (i, 0))],
        out_shape=[jax.ShapeDtypeStruct((e_pad, H), F32),
                   jax.ShapeDtypeStruct((e_pad // 8, 128), F32)],
    )(gs, gd, ps, pd, ea, wea, wd2, we2, be2, wa, ba, wc1, bc1, wc2)


def _node_body(h_ref, p_ref, a0m_ref, a1m_ref, a2m_ref, a3m_ref, a0r_ref,
               a1r_ref, a2r_ref, a3r_ref, wn1h, wn1a, bn1, wn2, bn2,
               ho_ref, po_ref):
    h = h_ref[...]
    aggm = (a0m_ref[...] + a1m_ref[...]) + (a2m_ref[...] + a3m_ref[...])
    agg16 = (a0r_ref[...] + a1r_ref[...]) + (a2r_ref[...] + a3r_ref[...])
    lane = lax.broadcasted_iota(jnp.int32, (NB, PW), 1)
    deg = jnp.sum(agg16 * (lane == 3).astype(F32), axis=1, keepdims=True)
    relc = jnp.where(lane < 3, agg16, 0.0)
    u = _silu(_dot(h, wn1h[...]) + _dot(aggm, wn1a[...]) + bn1[...])
    ho_ref[...] = h + _dot(u, wn2[...]) + bn2[...]
    po_ref[...] = p_ref[...] + relc / (deg + 1.0)


def _node_call(h, pos16, ams, ars, wn1h, wn1a, bn1, wn2, bn2):
    n = h.shape[0]
    return pl.pallas_call(
        _node_body,
        grid=(n // NB,),
        in_specs=[pl.BlockSpec((NB, H), lambda i: (i, 0)),
                  pl.BlockSpec((NB, PW), lambda i: (i, 0)),
                  pl.BlockSpec((NB, H), lambda i: (i, 0)),
                  pl.BlockSpec((NB, H), lambda i: (i, 0)),
                  pl.BlockSpec((NB, H), lambda i: (i, 0)),
                  pl.BlockSpec((NB, H), lambda i: (i, 0)),
                  pl.BlockSpec((NB, PW), lambda i: (i, 0)),
                  pl.BlockSpec((NB, PW), lambda i: (i, 0)),
                  pl.BlockSpec((NB, PW), lambda i: (i, 0)),
                  pl.BlockSpec((NB, PW), lambda i: (i, 0)),
                  pl.BlockSpec((H, H), lambda i: (0, 0)),
                  pl.BlockSpec((H, H), lambda i: (0, 0)),
                  pl.BlockSpec((1, H), lambda i: (0, 0)),
                  pl.BlockSpec((H, H), lambda i: (0, 0)),
                  pl.BlockSpec((1, H), lambda i: (0, 0))],
        out_specs=[pl.BlockSpec((NB, H), lambda i: (i, 0)),
                   pl.BlockSpec((NB, PW), lambda i: (i, 0))],
        out_shape=[jax.ShapeDtypeStruct((n, H), F32),
                   jax.ShapeDtypeStruct((n, PW), F32)],
    )(h, pos16, *ams, *ars, wn1h, wn1a, bn1, wn2, bn2)


def _final_body(h_ref, wo, bo, p_ref, p0_ref, t_ref, c_ref):
    t_ref[...] = _dot(h_ref[...], wo[...]) + bo[...]
    c_ref[...] = p_ref[...] - p0_ref[...]


def _final_call(h, wo, bo, pos16, pos016):
    n = h.shape[0]
    return pl.pallas_call(
        _final_body,
        grid=(n // NB,),
        in_specs=[pl.BlockSpec((NB, H), lambda i: (i, 0)),
                  pl.BlockSpec((H, NT), lambda i: (0, 0)),
                  pl.BlockSpec((1, NT), lambda i: (0, 0)),
                  pl.BlockSpec((NB, PW), lambda i: (i, 0)),
                  pl.BlockSpec((NB, PW), lambda i: (i, 0))],
        out_specs=[pl.BlockSpec((NB, NT), lambda i: (i, 0)),
                   pl.BlockSpec((NB, PW), lambda i: (i, 0))],
        out_shape=[jax.ShapeDtypeStruct((n, NT), F32),
                   jax.ShapeDtypeStruct((n, PW), F32)],
    )(h, wo, bo, pos16, pos016)


@functools.lru_cache(maxsize=None)
def _gather_kernel(n_rows, e_pad):
    chunks = e_pad // (NWORK * CH)
    per_w = chunks * CH

    @functools.partial(
        pl.kernel,
        out_type=(jax.ShapeDtypeStruct((e_pad, H), F32),
                  jax.ShapeDtypeStruct((e_pad, H), F32),
                  jax.ShapeDtypeStruct((e_pad, PW), F32),
                  jax.ShapeDtypeStruct((e_pad, PW), F32)),
        mesh=_mesh(),
        scratch_types=[pltpu.VMEM((chunks, CH), jnp.int32),
                       pltpu.VMEM((chunks, CH), jnp.int32),
                       pltpu.VMEM((2, CH, H), F32),
                       pltpu.VMEM((2, CH, H), F32),
                       pltpu.VMEM((2, CH, PW), F32),
                       pltpu.VMEM((2, CH, PW), F32),
                       pltpu.SemaphoreType.DMA,
                       pltpu.SemaphoreType.DMA,
                       pltpu.SemaphoreType.DMA,
                       pltpu.SemaphoreType.DMA],
        compiler_params=pltpu.CompilerParams(use_tc_tiling_on_sc=False))
    def k(tsrc, tdst, pos, srci, dsti, gs, gd, qs, qd, idx_s_ref, idx_d_ref,
          row_s, row_d, pr_s, pr_d, sem_g0, sem_g1, sem_w0, sem_w1):
        w = lax.axis_index("c") * NSUB + lax.axis_index("s")
        pltpu.sync_copy(srci.at[pl.ds(w * chunks, chunks)], idx_s_ref)
        pltpu.sync_copy(dsti.at[pl.ds(w * chunks, chunks)], idx_d_ref)
        sem_g = [sem_g0, sem_g1]
        sem_w = [sem_w0, sem_w1]

        def body(i, carry):
            base = w * per_w + i * CH

            def run(p):
                @pl.when(i >= 2)
                def _():
                    pltpu.make_async_copy(
                        row_s.at[p], gs.at[pl.ds(base, CH)],
                        sem_w[p]).wait()
                    pltpu.make_async_copy(
                        row_d.at[p], gd.at[pl.ds(base, CH)],
                        sem_w[p]).wait()
                    pltpu.make_async_copy(
                        pr_s.at[p], qs.at[pl.ds(base, CH)],
                        sem_w[p]).wait()
                    pltpu.make_async_copy(
                        pr_d.at[p], qd.at[pl.ds(base, CH)],
                        sem_w[p]).wait()
                cps = [
                    pltpu.async_copy(tsrc.at[idx_s_ref.at[i]], row_s.at[p],
                                     sem_g[p]),
                    pltpu.async_copy(tdst.at[idx_d_ref.at[i]], row_d.at[p],
                                     sem_g[p]),
                    pltpu.async_copy(pos.at[idx_s_ref.at[i]], pr_s.at[p],
                                     sem_g[p]),
                    pltpu.async_copy(pos.at[idx_d_ref.at[i]], pr_d.at[p],
                                     sem_g[p]),
                ]
                for c in cps:
                    c.wait()
                pltpu.async_copy(row_s.at[p], gs.at[pl.ds(base, CH)],
                                 sem_w[p])
                pltpu.async_copy(row_d.at[p], gd.at[pl.ds(base, CH)],
                                 sem_w[p])
                pltpu.async_copy(pr_s.at[p], qs.at[pl.ds(base, CH)],
                                 sem_w[p])
                pltpu.async_copy(pr_d.at[p], qd.at[pl.ds(base, CH)],
                                 sem_w[p])

            @pl.when(i % 2 == 0)
            def _():
                run(0)

            @pl.when(i % 2 == 1)
            def _():
                run(1)

            return carry

        lax.fori_loop(0, chunks, body, 0)
        for p in range(2):
            pltpu.make_async_copy(row_s.at[p], gs.at[pl.ds(0, CH)],
                                  sem_w[p]).wait()
            pltpu.make_async_copy(row_d.at[p], gd.at[pl.ds(0, CH)],
                                  sem_w[p]).wait()
            pltpu.make_async_copy(pr_s.at[p], qs.at[pl.ds(0, CH)],
                                  sem_w[p]).wait()
            pltpu.make_async_copy(pr_d.at[p], qd.at[pl.ds(0, CH)],
                                  sem_w[p]).wait()

    return k


@functools.lru_cache(maxsize=None)
def _scatter_kernel(n_rows, e_pad):
    chunks = e_pad // (NWORK * CH)
    per_w = chunks * CH
    rpt = n_rows // NSUB
    oc = 5
    ocs = rpt // oc

    @functools.partial(
        pl.kernel,
        out_type=(jax.ShapeDtypeStruct((NCORES, n_rows, H), F32),
                  jax.ShapeDtypeStruct((NCORES, n_rows, PW), F32)),
        mesh=_mesh(),
        scratch_types=[pltpu.VMEM_SHARED((n_rows, H), F32),
                       pltpu.VMEM_SHARED((n_rows, PW), F32),
                       pltpu.VMEM((2, CH), jnp.int32),
                       pltpu.VMEM((2, CH, H), F32),
                       pltpu.VMEM((2, CH, PW), F32),
                       pltpu.SemaphoreType.DMA,
                       pltpu.SemaphoreType.DMA,
                       pltpu.SemaphoreType.DMA,
                       pltpu.SemaphoreType.DMA],
        compiler_params=pltpu.CompilerParams(use_tc_tiling_on_sc=False))
    def k(mrm, mrr, dsti, zm, zr, outm, outr, accm, accr, idx, valm, valr,
          sem_a0, sem_a1, sem_l0, sem_l1):
        cid = lax.axis_index("c")
        sid = lax.axis_index("s")
        w = cid * NSUB + sid
        pltpu.sync_copy(zm, accm.at[pl.ds(sid * rpt, rpt)])
        pltpu.sync_copy(zr, accr.at[pl.ds(sid * rpt, rpt)])
        plsc.subcore_barrier()
        sem_a = [sem_a0, sem_a1]
        sem_l = [sem_l0, sem_l1]

        def body(i, carry):
            base = w * per_w + i * CH

            def run(p):
                @pl.when(i >= 2)
                def _():
                    pltpu.make_async_copy(valm.at[p], accm.at[idx.at[p]],
                                          sem_a[p]).wait()
                    pltpu.make_async_copy(valr.at[p], accr.at[idx.at[p]],
                                          sem_a[p]).wait()
                cps = [
                    pltpu.async_copy(dsti.at[w * chunks + i], idx.at[p],
                                     sem_l[p]),
                    pltpu.async_copy(mrm.at[pl.ds(base, CH)], valm.at[p],
                                     sem_l[p]),
                    pltpu.async_copy(mrr.at[pl.ds(base, CH)], valr.at[p],
                                     sem_l[p]),
                ]
                for c in cps:
                    c.wait()
                pltpu.async_copy(valm.at[p], accm.at[idx.at[p]], sem_a[p],
                                 add=True)
                pltpu.async_copy(valr.at[p], accr.at[idx.at[p]], sem_a[p],
                                 add=True)

            @pl.when(i % 2 == 0)
            def _():
                run(0)

            @pl.when(i % 2 == 1)
            def _():
                run(1)

            return carry

        lax.fori_loop(0, chunks, body, 0)
        for p in range(2):
            pltpu.make_async_copy(valm.at[p], accm.at[idx.at[p]],
                                  sem_a[p]).wait()
            pltpu.make_async_copy(valr.at[p], accr.at[idx.at[p]],
                                  sem_a[p]).wait()
        plsc.subcore_barrier()

        def flush(j, carry):
            s = sid * rpt + j * ocs
            pltpu.sync_copy(accm.at[pl.ds(s, ocs)],
                            outm.at[cid, pl.ds(s, ocs)])
            pltpu.sync_copy(accr.at[pl.ds(s, ocs)],
                            outr.at[cid, pl.ds(s, ocs)])
            return carry

        lax.fori_loop(0, oc, flush, 0)

    return k


def _egnn_layer(h, pos16, src_p, dst_p, ea, ne_real, zm, zr, p):
    n = h.shape[0]
    nrows = src_p.shape[0]
    e_half = (nrows // 2) * CH
    we1 = p['We1']
    tsrc, tdst = _pre_call(h, we1[:H], we1[H:2 * H], p['be1'].reshape(1, H))
    hrows = nrows // 2
    gath = _gather_kernel(n, e_half)
    scat = _scatter_kernel(n, e_half)
    halves = []
    for hf in range(2):
        src_h = src_p[hf * hrows:(hf + 1) * hrows]
        dst_h = dst_p[hf * hrows:(hf + 1) * hrows]
        halves.append((src_h, dst_h))
    gres = [gath(tsrc, tdst, pos16, s, d) for s, d in halves]
    mres = []
    for hf, (gs, gd, qs, qd) in enumerate(gres):
        qs = qs.reshape(e_half // 8, 128)
        qd = qd.reshape(e_half // 8, 128)
        mres.append(_edge_call(ne_real, hf * e_half, gs, gd, qs, qd, ea,
                               we1[2 * H + 1:], we1[2 * H].reshape(1, H),
                               p['We2'], p['be2'].reshape(1, H),
                               p['Wa'].reshape(1, H), p['ba'].reshape(1, 1),
                               p['Wc1'], p['bc1'].reshape(1, H),
                               p['Wc2'].reshape(1, H)))
    ams, ars = [], []
    for hf, (mrm, mrr) in enumerate(mres):
        pm, pr = scat(mrm, mrr.reshape(e_half, PW), halves[hf][1], zm, zr)
        ams.extend([pm[0], pm[1]])
        ars.extend([pr[0], pr[1]])
    wn1 = p['Wn1']
    return _node_call(h, pos16, ams, ars, wn1[:H], wn1[H:],
                      p['bn1'].reshape(1, H), p['Wn2'],
                      p['bn2'].reshape(1, H))


def _pad_edges(edge_index):
    e = edge_index.shape[1]
    e_pad = -(-e // (NWORK * CH)) * (NWORK * CH)
    pad = e_pad - e
    src = jnp.pad(edge_index[0].astype(jnp.int32), (0, pad)).reshape(-1, CH)
    dst = jnp.pad(edge_index[1].astype(jnp.int32), (0, pad)).reshape(-1, CH)
    return src, dst, e


def kernel(lig_x, lig_pos, lig_edge_index, lig_edge_attr, pocket_x,
           pocket_pos, pocket_edge_index, pocket_edge_attr, t, lig_batch,
           pocket_batch, L_We1, L_be1, L_We2, L_be2, L_Wa, L_ba, L_Wn1,
           L_bn1, L_Wn2, L_bn2, L_Wc1, L_bc1, L_Wc2, P_We1, P_be1, P_We2,
           P_be2, P_Wa, P_ba, P_Wn1, P_bn1, P_Wn2, P_bn2, P_Wc1, P_bc1,
           P_Wc2, W_emb_l, b_emb_l, W_emb_p, b_emb_p, Wt1, bt1, Wt2, bt2,
           Wp, bp, Wo, bo):
    pnames = ['We1', 'be1', 'We2', 'be2', 'Wa', 'ba', 'Wn1', 'bn1',
              'Wn2', 'bn2', 'Wc1', 'bc1', 'Wc2']
    lw = dict(zip(pnames, [L_We1, L_be1, L_We2, L_be2, L_Wa, L_ba, L_Wn1,
                           L_bn1, L_Wn2, L_bn2, L_Wc1, L_bc1, L_Wc2]))
    pw = dict(zip(pnames, [P_We1, P_be1, P_We2, P_be2, P_Wa, P_ba, P_Wn1,
                           P_bn1, P_Wn2, P_bn2, P_Wc1, P_bc1, P_Wc2]))

    n_p = pocket_x.shape[0]
    n_l = lig_x.shape[0]
    zm = jnp.zeros((n_p // NSUB, H), F32)
    zr = jnp.zeros((n_p // NSUB, PW), F32)

    hp = _emb_call(pocket_x, W_emb_p, b_emb_p.reshape(1, H))
    pp16 = jnp.pad(pocket_pos, ((0, 0), (0, PW - 3)))
    src_p, dst_p, ne_p = _pad_edges(pocket_edge_index)
    for i in range(P_We1.shape[0]):
        hp, pp16 = _egnn_layer(hp, pp16, src_p, dst_p, pocket_edge_attr,
                               ne_p, zm, zr,
                               {k: v[i] for k, v in pw.items()})

    psum = _pool_call(hp, pocket_batch.astype(jnp.int32).reshape(n_p, 1))
    tcond = _temb_cond_call(t.astype(jnp.int32).reshape(NBATCH, 1),
                            Wt1, bt1.reshape(1, H), Wt2, bt2.reshape(1, H),
                            psum, Wp, bp.reshape(1, H))

    h = _emb_lig_call(lig_x, lig_batch.astype(jnp.int32).reshape(n_l, 1),
                      W_emb_l, b_emb_l.reshape(1, H), tcond)
    pl16 = jnp.pad(lig_pos, ((0, 0), (0, PW - 3)))
    src_l, dst_l, ne_l = _pad_edges(lig_edge_index)
    pos16 = pl16
    for i in range(L_We1.shape[0]):
        h, pos16 = _egnn_layer(h, pos16, src_l, dst_l, lig_edge_attr,
                               ne_l, zm, zr,
                               {k: v[i] for k, v in lw.items()})

    type_pred, coord16 = _final_call(h, Wo, bo.reshape(1, NT), pos16, pl16)
    return type_pred, coord16[:, :3]
